# baseline (speedup 1.0000x reference)
import functools

import jax
import jax.numpy as jnp
from jax import lax
from jax.experimental import pallas as pl
from jax.experimental.pallas import tpu as pltpu
from jax.experimental.pallas import tpu_sc as plsc

N = 10000
E = 320000
NSUB = 16
NCORE = 2
CHUNK = 128
EPS = 20480
E_PAD = NSUB * EPS
CH = EPS // CHUNK
FH = 16
N_ACC = 10240
ZROWS = N_ACC // NSUB
CROWS = N // NSUB
CSPLIT = (128, 128, 128, 128, 113)

_PREC = jax.lax.Precision.HIGHEST


def _mesh():
    return plsc.VectorSubcoreMesh(core_axis_name="c", subcore_axis_name="s")


_SC_PARAMS = pltpu.CompilerParams(use_tc_tiling_on_sc=False)


def _deg_partials(dst3, ones_rows, zrows):
    @functools.partial(
        pl.kernel,
        out_type=jax.ShapeDtypeStruct((NCORE, N_ACC, 8), jnp.float32),
        mesh=_mesh(),
        scratch_types=[
            pltpu.VMEM((CH // 2, CHUNK), jnp.int32),
            pltpu.VMEM((CHUNK, 8), jnp.float32),
            pltpu.VMEM_SHARED((N_ACC, 8), jnp.float32),
            pltpu.SemaphoreType.DMA,
        ],
        compiler_params=_SC_PARAMS,
    )
    def deg_kernel(dst_hbm, ones_hbm, z_hbm, out_hbm, dstv, onesv, acc, sem):
        c = lax.axis_index("c")
        s = lax.axis_index("s")
        w = c * NSUB + s
        pltpu.sync_copy(z_hbm, acc.at[pl.ds(s * ZROWS, ZROWS)])
        pltpu.sync_copy(dst_hbm.at[w], dstv)
        pltpu.sync_copy(ones_hbm, onesv)
        plsc.subcore_barrier()

        @pl.loop(0, CH // 2)
        def _(j):
            pltpu.sync_copy(onesv, acc.at[dstv.at[j]], add=True)

        plsc.subcore_barrier()
        pltpu.sync_copy(acc.at[pl.ds(s * ZROWS, ZROWS)],
                        out_hbm.at[c, pl.ds(s * ZROWS, ZROWS)])

    return deg_kernel(dst3, ones_rows, zrows)


def _layer_sc(u0, qa, qb, d2, src3, dst3):
    @functools.partial(
        pl.kernel,
        out_type=(jax.ShapeDtypeStruct((NCORE, N_ACC, FH), jnp.float32),
                  jax.ShapeDtypeStruct((NCORE, N, FH), jnp.float32),
                  jax.ShapeDtypeStruct((NCORE, N, FH), jnp.float32)),
        mesh=_mesh(),
        scratch_types=[
            pltpu.VMEM((CH, CHUNK), jnp.int32),
            pltpu.VMEM((CH, CHUNK), jnp.int32),
            pltpu.VMEM((CHUNK, FH), jnp.float32),
            pltpu.VMEM((CHUNK, FH), jnp.float32),
            pltpu.VMEM((CHUNK, FH), jnp.float32),
            pltpu.VMEM((CHUNK, FH), jnp.float32),
            pltpu.VMEM((CHUNK, FH), jnp.float32),
            pltpu.VMEM((CHUNK, FH), jnp.float32),
            pltpu.VMEM_SHARED((N_ACC, FH), jnp.float32),
            pltpu.SemaphoreType.DMA,
            pltpu.SemaphoreType.DMA,
        ],
        compiler_params=_SC_PARAMS,
    )
    def layer_kernel(u0_hbm, qa_hbm, qb_hbm, d2_hbm, src_hbm, dst_hbm,
                     raw_hbm, u1_hbm, u2_hbm,
                     srcv, dstv, rows0, rows1, abuf, qbuf, dbuf, zbuf,
                     acc, sem0, sem1):
        c = lax.axis_index("c")
        s = lax.axis_index("s")

        @pl.loop(0, CHUNK)
        def _(i):
            zbuf[i, :] = jnp.zeros((FH,), jnp.float32)

        for z in range(ZROWS // CHUNK):
            pltpu.sync_copy(zbuf, acc.at[pl.ds(s * ZROWS + z * CHUNK, CHUNK)])
        pltpu.sync_copy(src_hbm.at[s], srcv)
        pltpu.sync_copy(dst_hbm.at[s], dstv)
        plsc.subcore_barrier()

        def hop(u_hbm):
            usrc = u_hbm.at[c]
            pltpu.async_copy(usrc.at[srcv.at[0]], rows0, sem0)

            @pl.loop(0, CH, step=2)
            def _(j):
                pltpu.async_copy(usrc.at[srcv.at[j + 1]], rows1, sem1)
                pltpu.make_async_copy(usrc.at[srcv.at[j]], rows0, sem0).wait()
                pltpu.sync_copy(rows0, acc.at[dstv.at[j]], add=True)

                @pl.when(j + 2 < CH)
                def _():
                    pltpu.async_copy(usrc.at[srcv.at[j + 2]], rows0, sem0)

                pltpu.make_async_copy(usrc.at[srcv.at[j + 1]], rows1,
                                      sem1).wait()
                pltpu.sync_copy(rows1, acc.at[dstv.at[j + 1]], add=True)

            plsc.subcore_barrier()

        def combine(q_hbm, unext_hbm):
            off = s * CROWS
            for sz in CSPLIT:
                pltpu.sync_copy(acc.at[pl.ds(off, sz)], abuf.at[pl.ds(0, sz)])
                pltpu.sync_copy(q_hbm.at[c, pl.ds(off, sz)],
                                qbuf.at[pl.ds(0, sz)])
                pltpu.sync_copy(d2_hbm.at[pl.ds(off, sz)],
                                dbuf.at[pl.ds(0, sz)])

                @pl.loop(0, sz)
                def _(i):
                    abuf[i, :] = qbuf[i, :] + dbuf[i, :] * abuf[i, :]

                pltpu.sync_copy(abuf.at[pl.ds(0, sz)],
                                unext_hbm.at[c, pl.ds(off, sz)])
                pltpu.sync_copy(zbuf.at[pl.ds(0, sz)], acc.at[pl.ds(off, sz)])
                off += sz
            plsc.subcore_barrier()

        hop(u0_hbm)
        combine(qa_hbm, u1_hbm)
        hop(u1_hbm)
        combine(qb_hbm, u2_hbm)
        hop(u2_hbm)

        for z in range(ZROWS // CHUNK):
            sl = pl.ds(s * ZROWS + z * CHUNK, CHUNK)
            pltpu.sync_copy(acc.at[sl], raw_hbm.at[c, sl])

    return layer_kernel(u0, qa, qb, d2, src3, dst3)[0]


def _tc_matmul(h, Wp):
    M, _ = h.shape
    _, Fo = Wp.shape

    def body(h_ref, w_ref, o_ref):
        o_ref[...] = jnp.dot(h_ref[...], w_ref[...],
                             preferred_element_type=jnp.float32,
                             precision=_PREC)

    return pl.pallas_call(
        body, out_shape=jax.ShapeDtypeStruct((M, Fo), jnp.float32))(h, Wp)


BROW = 2000
BGRID = N // BROW


def _split16(dis, P, F, k, rows):
    fh = F // 2
    blocks = []
    for c in range(2):
        blk = dis * P[:, k * F + c * fh:k * F + (c + 1) * fh]
        if fh < FH:
            blk = jnp.concatenate(
                [blk, jnp.zeros((rows, FH - fh), jnp.float32)], axis=1)
        blocks.append(blk)
    return jnp.stack(blocks)


def _tc_pre(degp, P1):
    def body(d_ref, p_ref, dis_ref, d2_ref, u0_ref, qa_ref, qb_ref):
        deg = d_ref[0, :, 0:1] + d_ref[1, :, 0:1]
        dis = jnp.where(deg > 0.0,
                        lax.rsqrt(jnp.maximum(deg, 1e-12)),
                        0.0)
        dis_ref[...] = dis
        d2_ref[...] = jnp.broadcast_to(dis * dis, (BROW, FH))
        P1 = p_ref[...]
        u0_ref[...] = _split16(dis, P1, 32, 3, BROW)
        qa_ref[...] = _split16(dis, P1, 32, 2, BROW)
        qb_ref[...] = _split16(dis, P1, 32, 1, BROW)

    return pl.pallas_call(
        body,
        grid=(BGRID,),
        in_specs=[
            pl.BlockSpec((2, BROW, 8), lambda i: (0, i, 0)),
            pl.BlockSpec((BROW, 128), lambda i: (i, 0)),
        ],
        out_specs=(
            pl.BlockSpec((BROW, 1), lambda i: (i, 0)),
            pl.BlockSpec((BROW, FH), lambda i: (i, 0)),
            pl.BlockSpec((2, BROW, FH), lambda i: (0, i, 0)),
            pl.BlockSpec((2, BROW, FH), lambda i: (0, i, 0)),
            pl.BlockSpec((2, BROW, FH), lambda i: (0, i, 0)),
        ),
        out_shape=(jax.ShapeDtypeStruct((N, 1), jnp.float32),
                   jax.ShapeDtypeStruct((N, FH), jnp.float32),
                   jax.ShapeDtypeStruct((NCORE, N, FH), jnp.float32),
                   jax.ShapeDtypeStruct((NCORE, N, FH), jnp.float32),
                   jax.ShapeDtypeStruct((NCORE, N, FH), jnp.float32)),
    )(degp, P1)


def _tc_layer(raw, P, dis, b, Wnext, F, Fn):
    fh = F // 2
    Fp, Fo = Wnext.shape

    def body(a_ref, p_ref, d_ref, b_ref, w_ref,
             pn_ref, u0_ref, qa_ref, qb_ref):
        dis = d_ref[...]
        t = jnp.concatenate(
            [p_ref[:, 0:fh] + dis * a_ref[0, :, 0:fh],
             p_ref[:, fh:F] + dis * a_ref[1, :, 0:fh]], axis=1)
        h = jnp.maximum(t + b_ref[...], 0.0)
        pn = jnp.dot(h, w_ref[...], preferred_element_type=jnp.float32,
                     precision=_PREC)
        pn_ref[...] = pn
        u0_ref[...] = _split16(dis, pn, Fn, 3, BROW)
        qa_ref[...] = _split16(dis, pn, Fn, 2, BROW)
        qb_ref[...] = _split16(dis, pn, Fn, 1, BROW)

    Fin = P.shape[1]
    return pl.pallas_call(
        body,
        grid=(BGRID,),
        in_specs=[
            pl.BlockSpec((2, BROW, FH), lambda i: (0, i, 0)),
            pl.BlockSpec((BROW, Fin), lambda i: (i, 0)),
            pl.BlockSpec((BROW, 1), lambda i: (i, 0)),
            pl.BlockSpec((1, Fp), lambda i: (0, 0)),
            pl.BlockSpec((Fp, Fo), lambda i: (0, 0)),
        ],
        out_specs=(
            pl.BlockSpec((BROW, Fo), lambda i: (i, 0)),
            pl.BlockSpec((2, BROW, FH), lambda i: (0, i, 0)),
            pl.BlockSpec((2, BROW, FH), lambda i: (0, i, 0)),
            pl.BlockSpec((2, BROW, FH), lambda i: (0, i, 0)),
        ),
        out_shape=(jax.ShapeDtypeStruct((N, Fo), jnp.float32),
                   jax.ShapeDtypeStruct((NCORE, N, FH), jnp.float32),
                   jax.ShapeDtypeStruct((NCORE, N, FH), jnp.float32),
                   jax.ShapeDtypeStruct((NCORE, N, FH), jnp.float32)),
    )(raw, P, dis, b.reshape(1, -1), Wnext)


def _tc_layer23(raw, P2, dis, b2, W3p):
    def body(a_ref, p_ref, d_ref, b_ref, w_ref,
             pn_ref, u0_ref, qa_ref, qb_ref):
        dis = d_ref[...]
        t = jnp.concatenate(
            [p_ref[:, 0:8] + dis * a_ref[0, :, 0:8],
             p_ref[:, 8:16] + dis * a_ref[1, :, 0:8]], axis=1)
        h = jnp.maximum(t + b_ref[...], 0.0)
        pn = jnp.dot(h, w_ref[...], preferred_element_type=jnp.float32,
                     precision=_PREC)
        pn_ref[...] = pn

        def dup(k):
            blk = jnp.concatenate(
                [dis * pn[:, 2 * k:2 * k + 2],
                 jnp.zeros((BROW, FH - 2), jnp.float32)], axis=1)
            return jnp.stack([blk, blk])

        u0_ref[...] = dup(3)
        qa_ref[...] = dup(2)
        qb_ref[...] = dup(1)

    return pl.pallas_call(
        body,
        grid=(BGRID,),
        in_specs=[
            pl.BlockSpec((2, BROW, FH), lambda i: (0, i, 0)),
            pl.BlockSpec((BROW, 64), lambda i: (i, 0)),
            pl.BlockSpec((BROW, 1), lambda i: (i, 0)),
            pl.BlockSpec((1, 16), lambda i: (0, 0)),
            pl.BlockSpec((16, 8), lambda i: (0, 0)),
        ],
        out_specs=(
            pl.BlockSpec((BROW, 8), lambda i: (i, 0)),
            pl.BlockSpec((2, BROW, FH), lambda i: (0, i, 0)),
            pl.BlockSpec((2, BROW, FH), lambda i: (0, i, 0)),
            pl.BlockSpec((2, BROW, FH), lambda i: (0, i, 0)),
        ),
        out_shape=(jax.ShapeDtypeStruct((N, 8), jnp.float32),
                   jax.ShapeDtypeStruct((NCORE, N, FH), jnp.float32),
                   jax.ShapeDtypeStruct((NCORE, N, FH), jnp.float32),
                   jax.ShapeDtypeStruct((NCORE, N, FH), jnp.float32)),
    )(raw, P2, dis, b2.reshape(1, -1), W3p)


def _tc_final(raw, P3, dis, b3):
    def body(a_ref, p_ref, d_ref, b_ref, o_ref):
        z = (p_ref[:, 0:2] + d_ref[...] * a_ref[0, :N, 0:2] + b_ref[...])
        m = jnp.max(z, axis=1, keepdims=True)
        lse = m + jnp.log(jnp.sum(jnp.exp(z - m), axis=1, keepdims=True))
        o_ref[...] = z - lse

    return pl.pallas_call(
        body, out_shape=jax.ShapeDtypeStruct((N, 2), jnp.float32),
    )(raw, P3, dis, b3.reshape(1, -1))


def kernel(x, edge_index, W1, b1, W2, b2, W3, b3):
    src = edge_index[0].astype(jnp.int32)
    dst = edge_index[1].astype(jnp.int32)
    pad = E_PAD - E
    src3 = jnp.concatenate([src, jnp.zeros((pad,), jnp.int32)]).reshape(
        NSUB, CH, CHUNK)
    dst3 = jnp.concatenate([dst, jnp.full((pad,), N, jnp.int32)]).reshape(
        NSUB, CH, CHUNK)
    src3d = src3.reshape(NSUB * 2, CH // 2, CHUNK)
    dst3d = dst3.reshape(NSUB * 2, CH // 2, CHUNK)
    del src3d

    ones8 = jnp.ones((CHUNK, 8), jnp.float32)
    z8 = jnp.zeros((ZROWS, 8), jnp.float32)

    W1p = jnp.concatenate([W1[i * 128:(i + 1) * 128] for i in range(4)], axis=1)
    W2p = jnp.concatenate([W2[i * 32:(i + 1) * 32] for i in range(4)], axis=1)
    W3p = jnp.concatenate([W3[i * 16:(i + 1) * 16] for i in range(4)], axis=1)

    degp = _deg_partials(dst3d, ones8, z8)
    P1 = _tc_matmul(x, W1p)
    dis, d2, u0, qa, qb = _tc_pre(degp, P1)

    raw1 = _layer_sc(u0, qa, qb, d2, src3, dst3)
    P2, u0, qa, qb = _tc_layer(raw1, P1, dis, b1, W2p, 32, 16)

    raw2 = _layer_sc(u0, qa, qb, d2, src3, dst3)
    P3, u0, qa, qb = _tc_layer23(raw2, P2, dis, b2, W3p)

    raw3 = _layer_sc(u0, qa, qb, d2, src3, dst3)
    return _tc_final(raw3, P3, dis, b3)


# speedup vs baseline: 1.1821x; 1.1821x over previous
import functools

import jax
import jax.numpy as jnp
from jax import lax
from jax.experimental import pallas as pl
from jax.experimental.pallas import tpu as pltpu
from jax.experimental.pallas import tpu_sc as plsc

N = 10000
E = 320000
NSUB = 16
NCORE = 2
CHUNK = 128
EPS = 20480
E_PAD = NSUB * EPS
CH = EPS // CHUNK
FH = 16
N_ACC = 10240
ZROWS = N_ACC // NSUB
CROWS = N // NSUB
CSPLIT = (128, 128, 128, 128, 113)
NBUF = 8
NDEPTH = 4

_PREC = jax.lax.Precision.HIGHEST


def _mesh():
    return plsc.VectorSubcoreMesh(core_axis_name="c", subcore_axis_name="s")


_SC_PARAMS = pltpu.CompilerParams(use_tc_tiling_on_sc=False)


def _deg_partials(dst3, ones_rows, zrows):
    @functools.partial(
        pl.kernel,
        out_type=jax.ShapeDtypeStruct((NCORE, N_ACC, 8), jnp.float32),
        mesh=_mesh(),
        scratch_types=[
            pltpu.VMEM((CH // 2, CHUNK), jnp.int32),
            pltpu.VMEM((CHUNK, 8), jnp.float32),
            pltpu.VMEM_SHARED((N_ACC, 8), jnp.float32),
            pltpu.SemaphoreType.DMA,
        ],
        compiler_params=_SC_PARAMS,
    )
    def deg_kernel(dst_hbm, ones_hbm, z_hbm, out_hbm, dstv, onesv, acc, sem):
        c = lax.axis_index("c")
        s = lax.axis_index("s")
        w = c * NSUB + s
        pltpu.sync_copy(z_hbm, acc.at[pl.ds(s * ZROWS, ZROWS)])
        pltpu.sync_copy(dst_hbm.at[w], dstv)
        pltpu.sync_copy(ones_hbm, onesv)
        plsc.subcore_barrier()

        @pl.loop(0, CH // 2)
        def _(j):
            pltpu.sync_copy(onesv, acc.at[dstv.at[j]], add=True)

        plsc.subcore_barrier()
        pltpu.sync_copy(acc.at[pl.ds(s * ZROWS, ZROWS)],
                        out_hbm.at[c, pl.ds(s * ZROWS, ZROWS)])

    return deg_kernel(dst3, ones_rows, zrows)


def _layer_sc(u0, qa, qb, d2, src3, dst3):
    @functools.partial(
        pl.kernel,
        out_type=(jax.ShapeDtypeStruct((NCORE, N_ACC, FH), jnp.float32),
                  jax.ShapeDtypeStruct((NCORE, N, FH), jnp.float32),
                  jax.ShapeDtypeStruct((NCORE, N, FH), jnp.float32)),
        mesh=_mesh(),
        scratch_types=[
            pltpu.VMEM((CH, CHUNK), jnp.int32),
            pltpu.VMEM((CH, CHUNK), jnp.int32),
            pltpu.VMEM((NBUF, CHUNK, FH), jnp.float32),
            pltpu.VMEM((CHUNK, FH), jnp.float32),
            pltpu.VMEM((CHUNK, FH), jnp.float32),
            pltpu.VMEM((CHUNK, FH), jnp.float32),
            pltpu.VMEM((CHUNK, FH), jnp.float32),
            pltpu.VMEM_SHARED((N_ACC, FH), jnp.float32),
            pltpu.SemaphoreType.DMA((NBUF,)),
            pltpu.SemaphoreType.DMA((NBUF,)),
        ],
        compiler_params=_SC_PARAMS,
    )
    def layer_kernel(u0_hbm, qa_hbm, qb_hbm, d2_hbm, src_hbm, dst_hbm,
                     raw_hbm, u1_hbm, u2_hbm,
                     srcv, dstv, rbuf, abuf, qbuf, dbuf, zbuf,
                     acc, gsem, ssem):
        c = lax.axis_index("c")
        s = lax.axis_index("s")

        @pl.loop(0, CHUNK)
        def _(i):
            zbuf[i, :] = jnp.zeros((FH,), jnp.float32)

        for z in range(ZROWS // CHUNK):
            pltpu.sync_copy(zbuf, acc.at[pl.ds(s * ZROWS + z * CHUNK, CHUNK)])
        pltpu.sync_copy(src_hbm.at[s], srcv)
        pltpu.sync_copy(dst_hbm.at[s], dstv)
        plsc.subcore_barrier()

        def hop(u_hbm):
            usrc = u_hbm.at[c]
            for kk in range(NDEPTH):
                pltpu.async_copy(usrc.at[srcv.at[kk]], rbuf.at[kk],
                                 gsem.at[kk])

            @pl.loop(0, CH, step=NBUF)
            def _(j):
                for l in range(NBUF):
                    k = j + l
                    pltpu.make_async_copy(usrc.at[srcv.at[k]], rbuf.at[l],
                                          gsem.at[l]).wait()
                    pltpu.async_copy(rbuf.at[l], acc.at[dstv.at[k]],
                                     ssem.at[l], add=True)
                    la = (l + NDEPTH) % NBUF
                    ka = k + NDEPTH

                    @pl.when(ka < CH)
                    def _():
                        @pl.when(ka >= NBUF)
                        def _():
                            pltpu.make_async_copy(
                                rbuf.at[la], acc.at[dstv.at[k]],
                                ssem.at[la]).wait()

                        pltpu.async_copy(usrc.at[srcv.at[ka]], rbuf.at[la],
                                         gsem.at[la])

            for l in range(NBUF):
                pltpu.make_async_copy(rbuf.at[l],
                                      acc.at[dstv.at[CH - NBUF + l]],
                                      ssem.at[l]).wait()
            plsc.subcore_barrier()

        def combine(q_hbm, unext_hbm):
            off = s * CROWS
            for sz in CSPLIT:
                pltpu.sync_copy(acc.at[pl.ds(off, sz)], abuf.at[pl.ds(0, sz)])
                pltpu.sync_copy(q_hbm.at[c, pl.ds(off, sz)],
                                qbuf.at[pl.ds(0, sz)])
                pltpu.sync_copy(d2_hbm.at[pl.ds(off, sz)],
                                dbuf.at[pl.ds(0, sz)])

                @pl.loop(0, sz)
                def _(i):
                    abuf[i, :] = qbuf[i, :] + dbuf[i, :] * abuf[i, :]

                pltpu.sync_copy(abuf.at[pl.ds(0, sz)],
                                unext_hbm.at[c, pl.ds(off, sz)])
                pltpu.sync_copy(zbuf.at[pl.ds(0, sz)], acc.at[pl.ds(off, sz)])
                off += sz
            plsc.subcore_barrier()

        hop(u0_hbm)
        combine(qa_hbm, u1_hbm)
        hop(u1_hbm)
        combine(qb_hbm, u2_hbm)
        hop(u2_hbm)

        for z in range(ZROWS // CHUNK):
            sl = pl.ds(s * ZROWS + z * CHUNK, CHUNK)
            pltpu.sync_copy(acc.at[sl], raw_hbm.at[c, sl])

    return layer_kernel(u0, qa, qb, d2, src3, dst3)[0]


def _tc_matmul(h, Wp):
    M, _ = h.shape
    _, Fo = Wp.shape

    def body(h_ref, w_ref, o_ref):
        o_ref[...] = jnp.dot(h_ref[...], w_ref[...],
                             preferred_element_type=jnp.float32,
                             precision=_PREC)

    return pl.pallas_call(
        body, out_shape=jax.ShapeDtypeStruct((M, Fo), jnp.float32))(h, Wp)


BROW = 2000
BGRID = N // BROW


def _split16(dis, P, F, k, rows):
    fh = F // 2
    blocks = []
    for c in range(2):
        blk = dis * P[:, k * F + c * fh:k * F + (c + 1) * fh]
        if fh < FH:
            blk = jnp.concatenate(
                [blk, jnp.zeros((rows, FH - fh), jnp.float32)], axis=1)
        blocks.append(blk)
    return jnp.stack(blocks)


def _tc_pre(degp, P1):
    def body(d_ref, p_ref, dis_ref, d2_ref, u0_ref, qa_ref, qb_ref):
        deg = d_ref[0, :, 0:1] + d_ref[1, :, 0:1]
        dis = jnp.where(deg > 0.0,
                        lax.rsqrt(jnp.maximum(deg, 1e-12)),
                        0.0)
        dis_ref[...] = dis
        d2_ref[...] = jnp.broadcast_to(dis * dis, (BROW, FH))
        P1 = p_ref[...]
        u0_ref[...] = _split16(dis, P1, 32, 3, BROW)
        qa_ref[...] = _split16(dis, P1, 32, 2, BROW)
        qb_ref[...] = _split16(dis, P1, 32, 1, BROW)

    return pl.pallas_call(
        body,
        grid=(BGRID,),
        in_specs=[
            pl.BlockSpec((2, BROW, 8), lambda i: (0, i, 0)),
            pl.BlockSpec((BROW, 128), lambda i: (i, 0)),
        ],
        out_specs=(
            pl.BlockSpec((BROW, 1), lambda i: (i, 0)),
            pl.BlockSpec((BROW, FH), lambda i: (i, 0)),
            pl.BlockSpec((2, BROW, FH), lambda i: (0, i, 0)),
            pl.BlockSpec((2, BROW, FH), lambda i: (0, i, 0)),
            pl.BlockSpec((2, BROW, FH), lambda i: (0, i, 0)),
        ),
        out_shape=(jax.ShapeDtypeStruct((N, 1), jnp.float32),
                   jax.ShapeDtypeStruct((N, FH), jnp.float32),
                   jax.ShapeDtypeStruct((NCORE, N, FH), jnp.float32),
                   jax.ShapeDtypeStruct((NCORE, N, FH), jnp.float32),
                   jax.ShapeDtypeStruct((NCORE, N, FH), jnp.float32)),
    )(degp, P1)


def _tc_layer(raw, P, dis, b, Wnext, F, Fn):
    fh = F // 2
    Fp, Fo = Wnext.shape

    def body(a_ref, p_ref, d_ref, b_ref, w_ref,
             pn_ref, u0_ref, qa_ref, qb_ref):
        dis = d_ref[...]
        t = jnp.concatenate(
            [p_ref[:, 0:fh] + dis * a_ref[0, :, 0:fh],
             p_ref[:, fh:F] + dis * a_ref[1, :, 0:fh]], axis=1)
        h = jnp.maximum(t + b_ref[...], 0.0)
        pn = jnp.dot(h, w_ref[...], preferred_element_type=jnp.float32,
                     precision=_PREC)
        pn_ref[...] = pn
        u0_ref[...] = _split16(dis, pn, Fn, 3, BROW)
        qa_ref[...] = _split16(dis, pn, Fn, 2, BROW)
        qb_ref[...] = _split16(dis, pn, Fn, 1, BROW)

    Fin = P.shape[1]
    return pl.pallas_call(
        body,
        grid=(BGRID,),
        in_specs=[
            pl.BlockSpec((2, BROW, FH), lambda i: (0, i, 0)),
            pl.BlockSpec((BROW, Fin), lambda i: (i, 0)),
            pl.BlockSpec((BROW, 1), lambda i: (i, 0)),
            pl.BlockSpec((1, Fp), lambda i: (0, 0)),
            pl.BlockSpec((Fp, Fo), lambda i: (0, 0)),
        ],
        out_specs=(
            pl.BlockSpec((BROW, Fo), lambda i: (i, 0)),
            pl.BlockSpec((2, BROW, FH), lambda i: (0, i, 0)),
            pl.BlockSpec((2, BROW, FH), lambda i: (0, i, 0)),
            pl.BlockSpec((2, BROW, FH), lambda i: (0, i, 0)),
        ),
        out_shape=(jax.ShapeDtypeStruct((N, Fo), jnp.float32),
                   jax.ShapeDtypeStruct((NCORE, N, FH), jnp.float32),
                   jax.ShapeDtypeStruct((NCORE, N, FH), jnp.float32),
                   jax.ShapeDtypeStruct((NCORE, N, FH), jnp.float32)),
    )(raw, P, dis, b.reshape(1, -1), Wnext)


def _tc_layer23(raw, P2, dis, b2, W3p):
    def body(a_ref, p_ref, d_ref, b_ref, w_ref,
             pn_ref, u0_ref, qa_ref, qb_ref):
        dis = d_ref[...]
        t = jnp.concatenate(
            [p_ref[:, 0:8] + dis * a_ref[0, :, 0:8],
             p_ref[:, 8:16] + dis * a_ref[1, :, 0:8]], axis=1)
        h = jnp.maximum(t + b_ref[...], 0.0)
        pn = jnp.dot(h, w_ref[...], preferred_element_type=jnp.float32,
                     precision=_PREC)
        pn_ref[...] = pn

        def dup(k):
            blk = jnp.concatenate(
                [dis * pn[:, 2 * k:2 * k + 2],
                 jnp.zeros((BROW, FH - 2), jnp.float32)], axis=1)
            return jnp.stack([blk, blk])

        u0_ref[...] = dup(3)
        qa_ref[...] = dup(2)
        qb_ref[...] = dup(1)

    return pl.pallas_call(
        body,
        grid=(BGRID,),
        in_specs=[
            pl.BlockSpec((2, BROW, FH), lambda i: (0, i, 0)),
            pl.BlockSpec((BROW, 64), lambda i: (i, 0)),
            pl.BlockSpec((BROW, 1), lambda i: (i, 0)),
            pl.BlockSpec((1, 16), lambda i: (0, 0)),
            pl.BlockSpec((16, 8), lambda i: (0, 0)),
        ],
        out_specs=(
            pl.BlockSpec((BROW, 8), lambda i: (i, 0)),
            pl.BlockSpec((2, BROW, FH), lambda i: (0, i, 0)),
            pl.BlockSpec((2, BROW, FH), lambda i: (0, i, 0)),
            pl.BlockSpec((2, BROW, FH), lambda i: (0, i, 0)),
        ),
        out_shape=(jax.ShapeDtypeStruct((N, 8), jnp.float32),
                   jax.ShapeDtypeStruct((NCORE, N, FH), jnp.float32),
                   jax.ShapeDtypeStruct((NCORE, N, FH), jnp.float32),
                   jax.ShapeDtypeStruct((NCORE, N, FH), jnp.float32)),
    )(raw, P2, dis, b2.reshape(1, -1), W3p)


def _tc_final(raw, P3, dis, b3):
    def body(a_ref, p_ref, d_ref, b_ref, o_ref):
        z = (p_ref[:, 0:2] + d_ref[...] * a_ref[0, :N, 0:2] + b_ref[...])
        m = jnp.max(z, axis=1, keepdims=True)
        lse = m + jnp.log(jnp.sum(jnp.exp(z - m), axis=1, keepdims=True))
        o_ref[...] = z - lse

    return pl.pallas_call(
        body, out_shape=jax.ShapeDtypeStruct((N, 2), jnp.float32),
    )(raw, P3, dis, b3.reshape(1, -1))


def kernel(x, edge_index, W1, b1, W2, b2, W3, b3):
    src = edge_index[0].astype(jnp.int32)
    dst = edge_index[1].astype(jnp.int32)
    pad = E_PAD - E
    src3 = jnp.concatenate([src, jnp.zeros((pad,), jnp.int32)]).reshape(
        NSUB, CH, CHUNK)
    dst3 = jnp.concatenate([dst, jnp.full((pad,), N, jnp.int32)]).reshape(
        NSUB, CH, CHUNK)
    src3d = src3.reshape(NSUB * 2, CH // 2, CHUNK)
    dst3d = dst3.reshape(NSUB * 2, CH // 2, CHUNK)
    del src3d

    ones8 = jnp.ones((CHUNK, 8), jnp.float32)
    z8 = jnp.zeros((ZROWS, 8), jnp.float32)

    W1p = jnp.concatenate([W1[i * 128:(i + 1) * 128] for i in range(4)], axis=1)
    W2p = jnp.concatenate([W2[i * 32:(i + 1) * 32] for i in range(4)], axis=1)
    W3p = jnp.concatenate([W3[i * 16:(i + 1) * 16] for i in range(4)], axis=1)

    degp = _deg_partials(dst3d, ones8, z8)
    P1 = _tc_matmul(x, W1p)
    dis, d2, u0, qa, qb = _tc_pre(degp, P1)

    raw1 = _layer_sc(u0, qa, qb, d2, src3, dst3)
    P2, u0, qa, qb = _tc_layer(raw1, P1, dis, b1, W2p, 32, 16)

    raw2 = _layer_sc(u0, qa, qb, d2, src3, dst3)
    P3, u0, qa, qb = _tc_layer23(raw2, P2, dis, b2, W3p)

    raw3 = _layer_sc(u0, qa, qb, d2, src3, dst3)
    return _tc_final(raw3, P3, dis, b3)


# speedup vs baseline: 1.1921x; 1.0085x over previous
import functools

import jax
import jax.numpy as jnp
from jax import lax
from jax.experimental import pallas as pl
from jax.experimental.pallas import tpu as pltpu
from jax.experimental.pallas import tpu_sc as plsc

N = 10000
E = 320000
NSUB = 16
NCORE = 2
CHUNK = 128
EPS = 20480
E_PAD = NSUB * EPS
CH = EPS // CHUNK
FH = 16
N_ACC = 10240
ZROWS = N_ACC // NSUB
CROWS = N // NSUB
CSPLIT = (128, 128, 128, 128, 113)
NBUF = 8
NDEPTH = 4

_PREC = jax.lax.Precision.HIGHEST


def _mesh():
    return plsc.VectorSubcoreMesh(core_axis_name="c", subcore_axis_name="s")


_SC_PARAMS = pltpu.CompilerParams(use_tc_tiling_on_sc=False)


def _deg_partials(dst3, ones_rows, zrows):
    @functools.partial(
        pl.kernel,
        out_type=jax.ShapeDtypeStruct((NCORE, N_ACC, 8), jnp.float32),
        mesh=_mesh(),
        scratch_types=[
            pltpu.VMEM((CH // 2, CHUNK), jnp.int32),
            pltpu.VMEM((CHUNK, 8), jnp.float32),
            pltpu.VMEM_SHARED((N_ACC, 8), jnp.float32),
            pltpu.SemaphoreType.DMA,
        ],
        compiler_params=_SC_PARAMS,
    )
    def deg_kernel(dst_hbm, ones_hbm, z_hbm, out_hbm, dstv, onesv, acc, sem):
        c = lax.axis_index("c")
        s = lax.axis_index("s")
        w = c * NSUB + s
        pltpu.sync_copy(z_hbm, acc.at[pl.ds(s * ZROWS, ZROWS)])
        pltpu.sync_copy(dst_hbm.at[w], dstv)
        pltpu.sync_copy(ones_hbm, onesv)
        plsc.subcore_barrier()

        @pl.loop(0, CH // 2)
        def _(j):
            pltpu.sync_copy(onesv, acc.at[dstv.at[j]], add=True)

        plsc.subcore_barrier()
        pltpu.sync_copy(acc.at[pl.ds(s * ZROWS, ZROWS)],
                        out_hbm.at[c, pl.ds(s * ZROWS, ZROWS)])

    return deg_kernel(dst3, ones_rows, zrows)


def _layer_sc(u0, qa, qb, d2, src3, dst3):
    @functools.partial(
        pl.kernel,
        out_type=(jax.ShapeDtypeStruct((NCORE, N_ACC, FH), jnp.float32),
                  jax.ShapeDtypeStruct((NCORE, N, FH), jnp.float32),
                  jax.ShapeDtypeStruct((NCORE, N, FH), jnp.float32)),
        mesh=_mesh(),
        scratch_types=[
            pltpu.VMEM((CH, CHUNK), jnp.int32),
            pltpu.VMEM((CH, CHUNK), jnp.int32),
            pltpu.VMEM((NBUF, CHUNK, FH), jnp.float32),
            pltpu.VMEM((CHUNK, FH), jnp.float32),
            pltpu.VMEM((CHUNK, FH), jnp.float32),
            pltpu.VMEM((CHUNK, FH), jnp.float32),
            pltpu.VMEM((CHUNK, FH), jnp.float32),
            pltpu.VMEM_SHARED((N_ACC, FH), jnp.float32),
            pltpu.SemaphoreType.DMA((NBUF,)),
            pltpu.SemaphoreType.DMA((NBUF,)),
        ],
        compiler_params=_SC_PARAMS,
    )
    def layer_kernel(u0_hbm, qa_hbm, qb_hbm, d2_hbm, src_hbm, dst_hbm,
                     raw_hbm, u1_hbm, u2_hbm,
                     srcv, dstv, rbuf, abuf, qbuf, dbuf, zbuf,
                     acc, gsem, ssem):
        c = lax.axis_index("c")
        s = lax.axis_index("s")

        pltpu.async_copy(src_hbm.at[s], srcv, gsem.at[0])
        pltpu.async_copy(dst_hbm.at[s], dstv, gsem.at[1])

        @pl.loop(0, CHUNK)
        def _(i):
            zbuf[i, :] = jnp.zeros((FH,), jnp.float32)

        for z in range(ZROWS // CHUNK):
            pltpu.async_copy(zbuf, acc.at[pl.ds(s * ZROWS + z * CHUNK, CHUNK)],
                             ssem.at[z])
        pltpu.make_async_copy(src_hbm.at[s], srcv, gsem.at[0]).wait()
        pltpu.make_async_copy(dst_hbm.at[s], dstv, gsem.at[1]).wait()
        for z in range(ZROWS // CHUNK):
            pltpu.make_async_copy(
                zbuf, acc.at[pl.ds(s * ZROWS + z * CHUNK, CHUNK)],
                ssem.at[z]).wait()
        plsc.subcore_barrier()

        def hop(u_hbm):
            usrc = u_hbm.at[c]
            for kk in range(NDEPTH):
                pltpu.async_copy(usrc.at[srcv.at[kk]], rbuf.at[kk],
                                 gsem.at[kk])

            @pl.loop(0, CH, step=NBUF)
            def _(j):
                for l in range(NBUF):
                    k = j + l
                    pltpu.make_async_copy(usrc.at[srcv.at[k]], rbuf.at[l],
                                          gsem.at[l]).wait()
                    pltpu.async_copy(rbuf.at[l], acc.at[dstv.at[k]],
                                     ssem.at[l], add=True)
                    la = (l + NDEPTH) % NBUF
                    ka = k + NDEPTH

                    @pl.when(ka < CH)
                    def _():
                        @pl.when(ka >= NBUF)
                        def _():
                            pltpu.make_async_copy(
                                rbuf.at[la], acc.at[dstv.at[k]],
                                ssem.at[la]).wait()

                        pltpu.async_copy(usrc.at[srcv.at[ka]], rbuf.at[la],
                                         gsem.at[la])

            for l in range(NBUF):
                pltpu.make_async_copy(rbuf.at[l],
                                      acc.at[dstv.at[CH - NBUF + l]],
                                      ssem.at[l]).wait()
            plsc.subcore_barrier()

        def combine(q_hbm, unext_hbm):
            off = s * CROWS
            for sz in CSPLIT:
                pltpu.sync_copy(acc.at[pl.ds(off, sz)], abuf.at[pl.ds(0, sz)])
                pltpu.sync_copy(q_hbm.at[c, pl.ds(off, sz)],
                                qbuf.at[pl.ds(0, sz)])
                pltpu.sync_copy(d2_hbm.at[pl.ds(off, sz)],
                                dbuf.at[pl.ds(0, sz)])

                @pl.loop(0, sz)
                def _(i):
                    abuf[i, :] = qbuf[i, :] + dbuf[i, :] * abuf[i, :]

                pltpu.sync_copy(abuf.at[pl.ds(0, sz)],
                                unext_hbm.at[c, pl.ds(off, sz)])
                pltpu.sync_copy(zbuf.at[pl.ds(0, sz)], acc.at[pl.ds(off, sz)])
                off += sz
            plsc.subcore_barrier()

        hop(u0_hbm)
        combine(qa_hbm, u1_hbm)
        hop(u1_hbm)
        combine(qb_hbm, u2_hbm)
        hop(u2_hbm)

        for z in range(ZROWS // CHUNK):
            sl = pl.ds(s * ZROWS + z * CHUNK, CHUNK)
            pltpu.sync_copy(acc.at[sl], raw_hbm.at[c, sl])

    return layer_kernel(u0, qa, qb, d2, src3, dst3)[0]


BROW = 2000
BGRID = N // BROW


def _split16(dis, P, F, k, rows):
    fh = F // 2
    blocks = []
    for c in range(2):
        blk = dis * P[:, k * F + c * fh:k * F + (c + 1) * fh]
        if fh < FH:
            blk = jnp.concatenate(
                [blk, jnp.zeros((rows, FH - fh), jnp.float32)], axis=1)
        blocks.append(blk)
    return jnp.stack(blocks)


def _tc_pre(degp, x, W1p):
    def body(d_ref, x_ref, w_ref, p_ref, dis_ref, d2_ref,
             u0_ref, qa_ref, qb_ref):
        deg = d_ref[0, :, 0:1] + d_ref[1, :, 0:1]
        dis = jnp.where(deg > 0.0,
                        lax.rsqrt(jnp.maximum(deg, 1e-12)),
                        0.0)
        dis_ref[...] = dis
        d2_ref[...] = jnp.broadcast_to(dis * dis, (BROW, FH))
        P1 = jnp.dot(x_ref[...], w_ref[...],
                     preferred_element_type=jnp.float32, precision=_PREC)
        p_ref[...] = P1
        u0_ref[...] = _split16(dis, P1, 32, 3, BROW)
        qa_ref[...] = _split16(dis, P1, 32, 2, BROW)
        qb_ref[...] = _split16(dis, P1, 32, 1, BROW)

    return pl.pallas_call(
        body,
        grid=(BGRID,),
        in_specs=[
            pl.BlockSpec((2, BROW, 8), lambda i: (0, i, 0)),
            pl.BlockSpec((BROW, 128), lambda i: (i, 0)),
            pl.BlockSpec((128, 128), lambda i: (0, 0)),
        ],
        out_specs=(
            pl.BlockSpec((BROW, 128), lambda i: (i, 0)),
            pl.BlockSpec((BROW, 1), lambda i: (i, 0)),
            pl.BlockSpec((BROW, FH), lambda i: (i, 0)),
            pl.BlockSpec((2, BROW, FH), lambda i: (0, i, 0)),
            pl.BlockSpec((2, BROW, FH), lambda i: (0, i, 0)),
            pl.BlockSpec((2, BROW, FH), lambda i: (0, i, 0)),
        ),
        out_shape=(jax.ShapeDtypeStruct((N, 128), jnp.float32),
                   jax.ShapeDtypeStruct((N, 1), jnp.float32),
                   jax.ShapeDtypeStruct((N, FH), jnp.float32),
                   jax.ShapeDtypeStruct((NCORE, N, FH), jnp.float32),
                   jax.ShapeDtypeStruct((NCORE, N, FH), jnp.float32),
                   jax.ShapeDtypeStruct((NCORE, N, FH), jnp.float32)),
    )(degp, x, W1p)


def _tc_layer(raw, P, dis, b, Wnext, F, Fn):
    fh = F // 2
    Fp, Fo = Wnext.shape

    def body(a_ref, p_ref, d_ref, b_ref, w_ref,
             pn_ref, u0_ref, qa_ref, qb_ref):
        dis = d_ref[...]
        t = jnp.concatenate(
            [p_ref[:, 0:fh] + dis * a_ref[0, :, 0:fh],
             p_ref[:, fh:F] + dis * a_ref[1, :, 0:fh]], axis=1)
        h = jnp.maximum(t + b_ref[...], 0.0)
        pn = jnp.dot(h, w_ref[...], preferred_element_type=jnp.float32,
                     precision=_PREC)
        pn_ref[...] = pn
        u0_ref[...] = _split16(dis, pn, Fn, 3, BROW)
        qa_ref[...] = _split16(dis, pn, Fn, 2, BROW)
        qb_ref[...] = _split16(dis, pn, Fn, 1, BROW)

    Fin = P.shape[1]
    return pl.pallas_call(
        body,
        grid=(BGRID,),
        in_specs=[
            pl.BlockSpec((2, BROW, FH), lambda i: (0, i, 0)),
            pl.BlockSpec((BROW, Fin), lambda i: (i, 0)),
            pl.BlockSpec((BROW, 1), lambda i: (i, 0)),
            pl.BlockSpec((1, Fp), lambda i: (0, 0)),
            pl.BlockSpec((Fp, Fo), lambda i: (0, 0)),
        ],
        out_specs=(
            pl.BlockSpec((BROW, Fo), lambda i: (i, 0)),
            pl.BlockSpec((2, BROW, FH), lambda i: (0, i, 0)),
            pl.BlockSpec((2, BROW, FH), lambda i: (0, i, 0)),
            pl.BlockSpec((2, BROW, FH), lambda i: (0, i, 0)),
        ),
        out_shape=(jax.ShapeDtypeStruct((N, Fo), jnp.float32),
                   jax.ShapeDtypeStruct((NCORE, N, FH), jnp.float32),
                   jax.ShapeDtypeStruct((NCORE, N, FH), jnp.float32),
                   jax.ShapeDtypeStruct((NCORE, N, FH), jnp.float32)),
    )(raw, P, dis, b.reshape(1, -1), Wnext)


def _tc_layer23(raw, P2, dis, b2, W3p):
    def body(a_ref, p_ref, d_ref, b_ref, w_ref,
             pn_ref, u0_ref, qa_ref, qb_ref):
        dis = d_ref[...]
        t = jnp.concatenate(
            [p_ref[:, 0:8] + dis * a_ref[0, :, 0:8],
             p_ref[:, 8:16] + dis * a_ref[1, :, 0:8]], axis=1)
        h = jnp.maximum(t + b_ref[...], 0.0)
        pn = jnp.dot(h, w_ref[...], preferred_element_type=jnp.float32,
                     precision=_PREC)
        pn_ref[...] = pn

        def dup(k):
            blk = jnp.concatenate(
                [dis * pn[:, 2 * k:2 * k + 2],
                 jnp.zeros((BROW, FH - 2), jnp.float32)], axis=1)
            return jnp.stack([blk, blk])

        u0_ref[...] = dup(3)
        qa_ref[...] = dup(2)
        qb_ref[...] = dup(1)

    return pl.pallas_call(
        body,
        grid=(BGRID,),
        in_specs=[
            pl.BlockSpec((2, BROW, FH), lambda i: (0, i, 0)),
            pl.BlockSpec((BROW, 64), lambda i: (i, 0)),
            pl.BlockSpec((BROW, 1), lambda i: (i, 0)),
            pl.BlockSpec((1, 16), lambda i: (0, 0)),
            pl.BlockSpec((16, 8), lambda i: (0, 0)),
        ],
        out_specs=(
            pl.BlockSpec((BROW, 8), lambda i: (i, 0)),
            pl.BlockSpec((2, BROW, FH), lambda i: (0, i, 0)),
            pl.BlockSpec((2, BROW, FH), lambda i: (0, i, 0)),
            pl.BlockSpec((2, BROW, FH), lambda i: (0, i, 0)),
        ),
        out_shape=(jax.ShapeDtypeStruct((N, 8), jnp.float32),
                   jax.ShapeDtypeStruct((NCORE, N, FH), jnp.float32),
                   jax.ShapeDtypeStruct((NCORE, N, FH), jnp.float32),
                   jax.ShapeDtypeStruct((NCORE, N, FH), jnp.float32)),
    )(raw, P2, dis, b2.reshape(1, -1), W3p)


def _tc_final(raw, P3, dis, b3):
    def body(a_ref, p_ref, d_ref, b_ref, o_ref):
        z = (p_ref[:, 0:2] + d_ref[...] * a_ref[0, :N, 0:2] + b_ref[...])
        m = jnp.max(z, axis=1, keepdims=True)
        lse = m + jnp.log(jnp.sum(jnp.exp(z - m), axis=1, keepdims=True))
        o_ref[...] = z - lse

    return pl.pallas_call(
        body, out_shape=jax.ShapeDtypeStruct((N, 2), jnp.float32),
    )(raw, P3, dis, b3.reshape(1, -1))


def kernel(x, edge_index, W1, b1, W2, b2, W3, b3):
    src = edge_index[0].astype(jnp.int32)
    dst = edge_index[1].astype(jnp.int32)
    pad = E_PAD - E
    src3 = jnp.concatenate([src, jnp.zeros((pad,), jnp.int32)]).reshape(
        NSUB, CH, CHUNK)
    dst3 = jnp.concatenate([dst, jnp.full((pad,), N, jnp.int32)]).reshape(
        NSUB, CH, CHUNK)
    src3d = src3.reshape(NSUB * 2, CH // 2, CHUNK)
    dst3d = dst3.reshape(NSUB * 2, CH // 2, CHUNK)
    del src3d

    ones8 = jnp.ones((CHUNK, 8), jnp.float32)
    z8 = jnp.zeros((ZROWS, 8), jnp.float32)

    W1p = jnp.concatenate([W1[i * 128:(i + 1) * 128] for i in range(4)], axis=1)
    W2p = jnp.concatenate([W2[i * 32:(i + 1) * 32] for i in range(4)], axis=1)
    W3p = jnp.concatenate([W3[i * 16:(i + 1) * 16] for i in range(4)], axis=1)

    degp = _deg_partials(dst3d, ones8, z8)
    P1, dis, d2, u0, qa, qb = _tc_pre(degp, x, W1p)

    raw1 = _layer_sc(u0, qa, qb, d2, src3, dst3)
    P2, u0, qa, qb = _tc_layer(raw1, P1, dis, b1, W2p, 32, 16)

    raw2 = _layer_sc(u0, qa, qb, d2, src3, dst3)
    P3, u0, qa, qb = _tc_layer23(raw2, P2, dis, b2, W3p)

    raw3 = _layer_sc(u0, qa, qb, d2, src3, dst3)
    return _tc_final(raw3, P3, dis, b3)


# speedup vs baseline: 1.2451x; 1.0445x over previous
import functools

import jax
import jax.numpy as jnp
from jax import lax
from jax.experimental import pallas as pl
from jax.experimental.pallas import tpu as pltpu
from jax.experimental.pallas import tpu_sc as plsc

N = 10000
E = 320000
NSUB = 16
NCORE = 2
CHUNK = 128
EPS = 20480
E_PAD = NSUB * EPS
CH = EPS // CHUNK
FH = 16
N_ACC = 10240
ZROWS = N_ACC // NSUB
CROWS = N // NSUB
CSPLIT = (128, 128, 128, 128, 113)
NBUF = 8
NDEPTH = 6

_PREC = jax.lax.Precision.HIGHEST


def _mesh():
    return plsc.VectorSubcoreMesh(core_axis_name="c", subcore_axis_name="s")


_SC_PARAMS = pltpu.CompilerParams(use_tc_tiling_on_sc=False)


def _deg_partials(dst3, ones_rows, zrows):
    @functools.partial(
        pl.kernel,
        out_type=jax.ShapeDtypeStruct((NCORE, N_ACC, 8), jnp.float32),
        mesh=_mesh(),
        scratch_types=[
            pltpu.VMEM((CH // 2, CHUNK), jnp.int32),
            pltpu.VMEM((CHUNK, 8), jnp.float32),
            pltpu.VMEM_SHARED((N_ACC, 8), jnp.float32),
            pltpu.SemaphoreType.DMA,
        ],
        compiler_params=_SC_PARAMS,
    )
    def deg_kernel(dst_hbm, ones_hbm, z_hbm, out_hbm, dstv, onesv, acc, sem):
        c = lax.axis_index("c")
        s = lax.axis_index("s")
        w = c * NSUB + s
        pltpu.sync_copy(z_hbm, acc.at[pl.ds(s * ZROWS, ZROWS)])
        pltpu.sync_copy(dst_hbm.at[w], dstv)
        pltpu.sync_copy(ones_hbm, onesv)
        plsc.subcore_barrier()

        @pl.loop(0, CH // 2)
        def _(j):
            pltpu.sync_copy(onesv, acc.at[dstv.at[j]], add=True)

        plsc.subcore_barrier()
        pltpu.sync_copy(acc.at[pl.ds(s * ZROWS, ZROWS)],
                        out_hbm.at[c, pl.ds(s * ZROWS, ZROWS)])

    return deg_kernel(dst3, ones_rows, zrows)


def _layer_sc(u0, qa, qb, d2, src3, dst3):
    @functools.partial(
        pl.kernel,
        out_type=(jax.ShapeDtypeStruct((NCORE, N_ACC, FH), jnp.float32),
                  jax.ShapeDtypeStruct((NCORE, N, FH), jnp.float32),
                  jax.ShapeDtypeStruct((NCORE, N, FH), jnp.float32)),
        mesh=_mesh(),
        scratch_types=[
            pltpu.VMEM((CH, CHUNK), jnp.int32),
            pltpu.VMEM((CH, CHUNK), jnp.int32),
            pltpu.VMEM((NBUF, CHUNK, FH), jnp.float32),
            pltpu.VMEM((CHUNK, FH), jnp.float32),
            pltpu.VMEM((CHUNK, FH), jnp.float32),
            pltpu.VMEM((CHUNK, FH), jnp.float32),
            pltpu.VMEM((CHUNK, FH), jnp.float32),
            pltpu.VMEM_SHARED((N_ACC, FH), jnp.float32),
            pltpu.SemaphoreType.DMA((NBUF,)),
            pltpu.SemaphoreType.DMA((NBUF,)),
        ],
        compiler_params=_SC_PARAMS,
    )
    def layer_kernel(u0_hbm, qa_hbm, qb_hbm, d2_hbm, src_hbm, dst_hbm,
                     raw_hbm, u1_hbm, u2_hbm,
                     srcv, dstv, rbuf, abuf, qbuf, dbuf, zbuf,
                     acc, gsem, ssem):
        c = lax.axis_index("c")
        s = lax.axis_index("s")

        pltpu.async_copy(src_hbm.at[s], srcv, gsem.at[0])
        pltpu.async_copy(dst_hbm.at[s], dstv, gsem.at[1])

        @pl.loop(0, CHUNK)
        def _(i):
            zbuf[i, :] = jnp.zeros((FH,), jnp.float32)

        for z in range(ZROWS // CHUNK):
            pltpu.async_copy(zbuf, acc.at[pl.ds(s * ZROWS + z * CHUNK, CHUNK)],
                             ssem.at[z])
        pltpu.make_async_copy(src_hbm.at[s], srcv, gsem.at[0]).wait()
        pltpu.make_async_copy(dst_hbm.at[s], dstv, gsem.at[1]).wait()
        for z in range(ZROWS // CHUNK):
            pltpu.make_async_copy(
                zbuf, acc.at[pl.ds(s * ZROWS + z * CHUNK, CHUNK)],
                ssem.at[z]).wait()
        plsc.subcore_barrier()

        def hop(u_hbm):
            usrc = u_hbm.at[c]
            for kk in range(NDEPTH):
                pltpu.async_copy(usrc.at[srcv.at[kk]], rbuf.at[kk],
                                 gsem.at[kk])

            @pl.loop(0, CH, step=NBUF)
            def _(j):
                for l in range(NBUF):
                    k = j + l
                    pltpu.make_async_copy(usrc.at[srcv.at[k]], rbuf.at[l],
                                          gsem.at[l]).wait()
                    pltpu.async_copy(rbuf.at[l], acc.at[dstv.at[k]],
                                     ssem.at[l], add=True)
                    la = (l + NDEPTH) % NBUF
                    ka = k + NDEPTH

                    @pl.when(ka < CH)
                    def _():
                        @pl.when(ka >= NBUF)
                        def _():
                            pltpu.make_async_copy(
                                rbuf.at[la], acc.at[dstv.at[k]],
                                ssem.at[la]).wait()

                        pltpu.async_copy(usrc.at[srcv.at[ka]], rbuf.at[la],
                                         gsem.at[la])

            for l in range(NBUF):
                pltpu.make_async_copy(rbuf.at[l],
                                      acc.at[dstv.at[CH - NBUF + l]],
                                      ssem.at[l]).wait()
            plsc.subcore_barrier()

        def combine(q_hbm, unext_hbm):
            off = s * CROWS
            for sz in CSPLIT:
                pltpu.sync_copy(acc.at[pl.ds(off, sz)], abuf.at[pl.ds(0, sz)])
                pltpu.sync_copy(q_hbm.at[c, pl.ds(off, sz)],
                                qbuf.at[pl.ds(0, sz)])
                pltpu.sync_copy(d2_hbm.at[pl.ds(off, sz)],
                                dbuf.at[pl.ds(0, sz)])

                @pl.loop(0, sz)
                def _(i):
                    abuf[i, :] = qbuf[i, :] + dbuf[i, :] * abuf[i, :]

                pltpu.sync_copy(abuf.at[pl.ds(0, sz)],
                                unext_hbm.at[c, pl.ds(off, sz)])
                pltpu.sync_copy(zbuf.at[pl.ds(0, sz)], acc.at[pl.ds(off, sz)])
                off += sz
            plsc.subcore_barrier()

        hop(u0_hbm)
        combine(qa_hbm, u1_hbm)
        hop(u1_hbm)
        combine(qb_hbm, u2_hbm)
        hop(u2_hbm)

        for z in range(ZROWS // CHUNK):
            sl = pl.ds(s * ZROWS + z * CHUNK, CHUNK)
            pltpu.sync_copy(acc.at[sl], raw_hbm.at[c, sl])

    return layer_kernel(u0, qa, qb, d2, src3, dst3)[0]


BROW = 2000
BGRID = N // BROW


def _split16(dis, P, F, k, rows):
    fh = F // 2
    blocks = []
    for c in range(2):
        blk = dis * P[:, k * F + c * fh:k * F + (c + 1) * fh]
        if fh < FH:
            blk = jnp.concatenate(
                [blk, jnp.zeros((rows, FH - fh), jnp.float32)], axis=1)
        blocks.append(blk)
    return jnp.stack(blocks)


def _tc_pre(degp, x, W1p):
    def body(d_ref, x_ref, w_ref, p_ref, dis_ref, d2_ref,
             u0_ref, qa_ref, qb_ref):
        deg = d_ref[0, :, 0:1] + d_ref[1, :, 0:1]
        dis = jnp.where(deg > 0.0,
                        lax.rsqrt(jnp.maximum(deg, 1e-12)),
                        0.0)
        dis_ref[...] = dis
        d2_ref[...] = jnp.broadcast_to(dis * dis, (BROW, FH))
        P1 = jnp.dot(x_ref[...], w_ref[...],
                     preferred_element_type=jnp.float32, precision=_PREC)
        p_ref[...] = P1
        u0_ref[...] = _split16(dis, P1, 32, 3, BROW)
        qa_ref[...] = _split16(dis, P1, 32, 2, BROW)
        qb_ref[...] = _split16(dis, P1, 32, 1, BROW)

    return pl.pallas_call(
        body,
        grid=(BGRID,),
        in_specs=[
            pl.BlockSpec((2, BROW, 8), lambda i: (0, i, 0)),
            pl.BlockSpec((BROW, 128), lambda i: (i, 0)),
            pl.BlockSpec((128, 128), lambda i: (0, 0)),
        ],
        out_specs=(
            pl.BlockSpec((BROW, 128), lambda i: (i, 0)),
            pl.BlockSpec((BROW, 1), lambda i: (i, 0)),
            pl.BlockSpec((BROW, FH), lambda i: (i, 0)),
            pl.BlockSpec((2, BROW, FH), lambda i: (0, i, 0)),
            pl.BlockSpec((2, BROW, FH), lambda i: (0, i, 0)),
            pl.BlockSpec((2, BROW, FH), lambda i: (0, i, 0)),
        ),
        out_shape=(jax.ShapeDtypeStruct((N, 128), jnp.float32),
                   jax.ShapeDtypeStruct((N, 1), jnp.float32),
                   jax.ShapeDtypeStruct((N, FH), jnp.float32),
                   jax.ShapeDtypeStruct((NCORE, N, FH), jnp.float32),
                   jax.ShapeDtypeStruct((NCORE, N, FH), jnp.float32),
                   jax.ShapeDtypeStruct((NCORE, N, FH), jnp.float32)),
    )(degp, x, W1p)


def _tc_layer(raw, P, dis, b, Wnext, F, Fn):
    fh = F // 2
    Fp, Fo = Wnext.shape

    def body(a_ref, p_ref, d_ref, b_ref, w_ref,
             pn_ref, u0_ref, qa_ref, qb_ref):
        dis = d_ref[...]
        t = jnp.concatenate(
            [p_ref[:, 0:fh] + dis * a_ref[0, :, 0:fh],
             p_ref[:, fh:F] + dis * a_ref[1, :, 0:fh]], axis=1)
        h = jnp.maximum(t + b_ref[...], 0.0)
        pn = jnp.dot(h, w_ref[...], preferred_element_type=jnp.float32,
                     precision=_PREC)
        pn_ref[...] = pn
        u0_ref[...] = _split16(dis, pn, Fn, 3, BROW)
        qa_ref[...] = _split16(dis, pn, Fn, 2, BROW)
        qb_ref[...] = _split16(dis, pn, Fn, 1, BROW)

    Fin = P.shape[1]
    return pl.pallas_call(
        body,
        grid=(BGRID,),
        in_specs=[
            pl.BlockSpec((2, BROW, FH), lambda i: (0, i, 0)),
            pl.BlockSpec((BROW, Fin), lambda i: (i, 0)),
            pl.BlockSpec((BROW, 1), lambda i: (i, 0)),
            pl.BlockSpec((1, Fp), lambda i: (0, 0)),
            pl.BlockSpec((Fp, Fo), lambda i: (0, 0)),
        ],
        out_specs=(
            pl.BlockSpec((BROW, Fo), lambda i: (i, 0)),
            pl.BlockSpec((2, BROW, FH), lambda i: (0, i, 0)),
            pl.BlockSpec((2, BROW, FH), lambda i: (0, i, 0)),
            pl.BlockSpec((2, BROW, FH), lambda i: (0, i, 0)),
        ),
        out_shape=(jax.ShapeDtypeStruct((N, Fo), jnp.float32),
                   jax.ShapeDtypeStruct((NCORE, N, FH), jnp.float32),
                   jax.ShapeDtypeStruct((NCORE, N, FH), jnp.float32),
                   jax.ShapeDtypeStruct((NCORE, N, FH), jnp.float32)),
    )(raw, P, dis, b.reshape(1, -1), Wnext)


def _tc_layer23(raw, P2, dis, b2, W3p):
    def body(a_ref, p_ref, d_ref, b_ref, w_ref,
             pn_ref, u0_ref, qa_ref, qb_ref):
        dis = d_ref[...]
        t = jnp.concatenate(
            [p_ref[:, 0:8] + dis * a_ref[0, :, 0:8],
             p_ref[:, 8:16] + dis * a_ref[1, :, 0:8]], axis=1)
        h = jnp.maximum(t + b_ref[...], 0.0)
        pn = jnp.dot(h, w_ref[...], preferred_element_type=jnp.float32,
                     precision=_PREC)
        pn_ref[...] = pn

        def dup(k):
            blk = jnp.concatenate(
                [dis * pn[:, 2 * k:2 * k + 2],
                 jnp.zeros((BROW, FH - 2), jnp.float32)], axis=1)
            return jnp.stack([blk, blk])

        u0_ref[...] = dup(3)
        qa_ref[...] = dup(2)
        qb_ref[...] = dup(1)

    return pl.pallas_call(
        body,
        grid=(BGRID,),
        in_specs=[
            pl.BlockSpec((2, BROW, FH), lambda i: (0, i, 0)),
            pl.BlockSpec((BROW, 64), lambda i: (i, 0)),
            pl.BlockSpec((BROW, 1), lambda i: (i, 0)),
            pl.BlockSpec((1, 16), lambda i: (0, 0)),
            pl.BlockSpec((16, 8), lambda i: (0, 0)),
        ],
        out_specs=(
            pl.BlockSpec((BROW, 8), lambda i: (i, 0)),
            pl.BlockSpec((2, BROW, FH), lambda i: (0, i, 0)),
            pl.BlockSpec((2, BROW, FH), lambda i: (0, i, 0)),
            pl.BlockSpec((2, BROW, FH), lambda i: (0, i, 0)),
        ),
        out_shape=(jax.ShapeDtypeStruct((N, 8), jnp.float32),
                   jax.ShapeDtypeStruct((NCORE, N, FH), jnp.float32),
                   jax.ShapeDtypeStruct((NCORE, N, FH), jnp.float32),
                   jax.ShapeDtypeStruct((NCORE, N, FH), jnp.float32)),
    )(raw, P2, dis, b2.reshape(1, -1), W3p)


def _tc_final(raw, P3, dis, b3):
    def body(a_ref, p_ref, d_ref, b_ref, o_ref):
        z = (p_ref[:, 0:2] + d_ref[...] * a_ref[0, :N, 0:2] + b_ref[...])
        m = jnp.max(z, axis=1, keepdims=True)
        lse = m + jnp.log(jnp.sum(jnp.exp(z - m), axis=1, keepdims=True))
        o_ref[...] = z - lse

    return pl.pallas_call(
        body, out_shape=jax.ShapeDtypeStruct((N, 2), jnp.float32),
    )(raw, P3, dis, b3.reshape(1, -1))


def kernel(x, edge_index, W1, b1, W2, b2, W3, b3):
    src = edge_index[0].astype(jnp.int32)
    dst = edge_index[1].astype(jnp.int32)
    pad = E_PAD - E
    src3 = jnp.concatenate([src, jnp.zeros((pad,), jnp.int32)]).reshape(
        NSUB, CH, CHUNK)
    dst3 = jnp.concatenate([dst, jnp.full((pad,), N, jnp.int32)]).reshape(
        NSUB, CH, CHUNK)
    src3d = src3.reshape(NSUB * 2, CH // 2, CHUNK)
    dst3d = dst3.reshape(NSUB * 2, CH // 2, CHUNK)
    del src3d

    ones8 = jnp.ones((CHUNK, 8), jnp.float32)
    z8 = jnp.zeros((ZROWS, 8), jnp.float32)

    W1p = jnp.concatenate([W1[i * 128:(i + 1) * 128] for i in range(4)], axis=1)
    W2p = jnp.concatenate([W2[i * 32:(i + 1) * 32] for i in range(4)], axis=1)
    W3p = jnp.concatenate([W3[i * 16:(i + 1) * 16] for i in range(4)], axis=1)

    degp = _deg_partials(dst3d, ones8, z8)
    P1, dis, d2, u0, qa, qb = _tc_pre(degp, x, W1p)

    raw1 = _layer_sc(u0, qa, qb, d2, src3, dst3)
    P2, u0, qa, qb = _tc_layer(raw1, P1, dis, b1, W2p, 32, 16)

    raw2 = _layer_sc(u0, qa, qb, d2, src3, dst3)
    P3, u0, qa, qb = _tc_layer23(raw2, P2, dis, b2, W3p)

    raw3 = _layer_sc(u0, qa, qb, d2, src3, dst3)
    return _tc_final(raw3, P3, dis, b3)


# speedup vs baseline: 1.2506x; 1.0044x over previous
import functools

import jax
import jax.numpy as jnp
from jax import lax
from jax.experimental import pallas as pl
from jax.experimental.pallas import tpu as pltpu
from jax.experimental.pallas import tpu_sc as plsc

N = 10000
E = 320000
NSUB = 16
NCORE = 2
CHUNK = 128
EPS = 20480
E_PAD = NSUB * EPS
CH = EPS // CHUNK
FH = 16
N_ACC = 10240
ZROWS = N_ACC // NSUB
CROWS = N // NSUB
CSPLIT = (128, 128, 128, 128, 113)
NBUF = 8
NDEPTH = 7

_PREC = jax.lax.Precision.HIGHEST


def _mesh():
    return plsc.VectorSubcoreMesh(core_axis_name="c", subcore_axis_name="s")


_SC_PARAMS = pltpu.CompilerParams(use_tc_tiling_on_sc=False)


def _deg_partials(dst3, ones_rows, zrows):
    @functools.partial(
        pl.kernel,
        out_type=jax.ShapeDtypeStruct((NCORE, N_ACC, 8), jnp.float32),
        mesh=_mesh(),
        scratch_types=[
            pltpu.VMEM((CH // 2, CHUNK), jnp.int32),
            pltpu.VMEM((CHUNK, 8), jnp.float32),
            pltpu.VMEM_SHARED((N_ACC, 8), jnp.float32),
            pltpu.SemaphoreType.DMA,
        ],
        compiler_params=_SC_PARAMS,
    )
    def deg_kernel(dst_hbm, ones_hbm, z_hbm, out_hbm, dstv, onesv, acc, sem):
        c = lax.axis_index("c")
        s = lax.axis_index("s")
        w = c * NSUB + s
        pltpu.sync_copy(z_hbm, acc.at[pl.ds(s * ZROWS, ZROWS)])
        pltpu.sync_copy(dst_hbm.at[w], dstv)
        pltpu.sync_copy(ones_hbm, onesv)
        plsc.subcore_barrier()

        @pl.loop(0, CH // 2)
        def _(j):
            pltpu.sync_copy(onesv, acc.at[dstv.at[j]], add=True)

        plsc.subcore_barrier()
        pltpu.sync_copy(acc.at[pl.ds(s * ZROWS, ZROWS)],
                        out_hbm.at[c, pl.ds(s * ZROWS, ZROWS)])

    return deg_kernel(dst3, ones_rows, zrows)


def _layer_sc(u0, qa, qb, d2, src3, dst3):
    @functools.partial(
        pl.kernel,
        out_type=(jax.ShapeDtypeStruct((NCORE, N_ACC, FH), jnp.float32),
                  jax.ShapeDtypeStruct((NCORE, N, FH), jnp.float32),
                  jax.ShapeDtypeStruct((NCORE, N, FH), jnp.float32)),
        mesh=_mesh(),
        scratch_types=[
            pltpu.VMEM((CH, CHUNK), jnp.int32),
            pltpu.VMEM((CH, CHUNK), jnp.int32),
            pltpu.VMEM((NBUF, CHUNK, FH), jnp.float32),
            pltpu.VMEM((CHUNK, FH), jnp.float32),
            pltpu.VMEM((CHUNK, FH), jnp.float32),
            pltpu.VMEM((CHUNK, FH), jnp.float32),
            pltpu.VMEM((CHUNK, FH), jnp.float32),
            pltpu.VMEM_SHARED((N_ACC, FH), jnp.float32),
            pltpu.SemaphoreType.DMA((NBUF,)),
            pltpu.SemaphoreType.DMA((NBUF,)),
        ],
        compiler_params=_SC_PARAMS,
    )
    def layer_kernel(u0_hbm, qa_hbm, qb_hbm, d2_hbm, src_hbm, dst_hbm,
                     raw_hbm, u1_hbm, u2_hbm,
                     srcv, dstv, rbuf, abuf, qbuf, dbuf, zbuf,
                     acc, gsem, ssem):
        c = lax.axis_index("c")
        s = lax.axis_index("s")

        pltpu.async_copy(src_hbm.at[s], srcv, gsem.at[0])
        pltpu.async_copy(dst_hbm.at[s], dstv, gsem.at[1])

        @pl.loop(0, CHUNK)
        def _(i):
            zbuf[i, :] = jnp.zeros((FH,), jnp.float32)

        for z in range(ZROWS // CHUNK):
            pltpu.async_copy(zbuf, acc.at[pl.ds(s * ZROWS + z * CHUNK, CHUNK)],
                             ssem.at[z])
        pltpu.make_async_copy(src_hbm.at[s], srcv, gsem.at[0]).wait()
        pltpu.make_async_copy(dst_hbm.at[s], dstv, gsem.at[1]).wait()
        for z in range(ZROWS // CHUNK):
            pltpu.make_async_copy(
                zbuf, acc.at[pl.ds(s * ZROWS + z * CHUNK, CHUNK)],
                ssem.at[z]).wait()
        plsc.subcore_barrier()

        def hop(u_hbm):
            usrc = u_hbm.at[c]
            for kk in range(NDEPTH):
                pltpu.async_copy(usrc.at[srcv.at[kk]], rbuf.at[kk],
                                 gsem.at[kk])

            @pl.loop(0, CH, step=NBUF)
            def _(j):
                for l in range(NBUF):
                    k = j + l
                    pltpu.make_async_copy(usrc.at[srcv.at[k]], rbuf.at[l],
                                          gsem.at[l]).wait()
                    pltpu.async_copy(rbuf.at[l], acc.at[dstv.at[k]],
                                     ssem.at[l], add=True)
                    la = (l + NDEPTH) % NBUF
                    ka = k + NDEPTH

                    @pl.when(ka < CH)
                    def _():
                        @pl.when(ka >= NBUF)
                        def _():
                            pltpu.make_async_copy(
                                rbuf.at[la], acc.at[dstv.at[k]],
                                ssem.at[la]).wait()

                        pltpu.async_copy(usrc.at[srcv.at[ka]], rbuf.at[la],
                                         gsem.at[la])

            for l in range(NBUF):
                pltpu.make_async_copy(rbuf.at[l],
                                      acc.at[dstv.at[CH - NBUF + l]],
                                      ssem.at[l]).wait()
            plsc.subcore_barrier()

        def combine(q_hbm, unext_hbm):
            off = s * CROWS
            for sz in CSPLIT:
                pltpu.sync_copy(acc.at[pl.ds(off, sz)], abuf.at[pl.ds(0, sz)])
                pltpu.sync_copy(q_hbm.at[c, pl.ds(off, sz)],
                                qbuf.at[pl.ds(0, sz)])
                pltpu.sync_copy(d2_hbm.at[pl.ds(off, sz)],
                                dbuf.at[pl.ds(0, sz)])

                @pl.loop(0, sz)
                def _(i):
                    abuf[i, :] = qbuf[i, :] + dbuf[i, :] * abuf[i, :]

                pltpu.sync_copy(abuf.at[pl.ds(0, sz)],
                                unext_hbm.at[c, pl.ds(off, sz)])
                pltpu.sync_copy(zbuf.at[pl.ds(0, sz)], acc.at[pl.ds(off, sz)])
                off += sz
            plsc.subcore_barrier()

        hop(u0_hbm)
        combine(qa_hbm, u1_hbm)
        hop(u1_hbm)
        combine(qb_hbm, u2_hbm)
        hop(u2_hbm)

        for z in range(ZROWS // CHUNK):
            sl = pl.ds(s * ZROWS + z * CHUNK, CHUNK)
            pltpu.sync_copy(acc.at[sl], raw_hbm.at[c, sl])

    return layer_kernel(u0, qa, qb, d2, src3, dst3)[0]


BROW = 2000
BGRID = N // BROW


def _split16(dis, P, F, k, rows):
    fh = F // 2
    blocks = []
    for c in range(2):
        blk = dis * P[:, k * F + c * fh:k * F + (c + 1) * fh]
        if fh < FH:
            blk = jnp.concatenate(
                [blk, jnp.zeros((rows, FH - fh), jnp.float32)], axis=1)
        blocks.append(blk)
    return jnp.stack(blocks)


def _tc_pre(degp, x, W1p):
    def body(d_ref, x_ref, w_ref, p_ref, dis_ref, d2_ref,
             u0_ref, qa_ref, qb_ref):
        deg = d_ref[0, :, 0:1] + d_ref[1, :, 0:1]
        dis = jnp.where(deg > 0.0,
                        lax.rsqrt(jnp.maximum(deg, 1e-12)),
                        0.0)
        dis_ref[...] = dis
        d2_ref[...] = jnp.broadcast_to(dis * dis, (BROW, FH))
        P1 = jnp.dot(x_ref[...], w_ref[...],
                     preferred_element_type=jnp.float32, precision=_PREC)
        p_ref[...] = P1
        u0_ref[...] = _split16(dis, P1, 32, 3, BROW)
        qa_ref[...] = _split16(dis, P1, 32, 2, BROW)
        qb_ref[...] = _split16(dis, P1, 32, 1, BROW)

    return pl.pallas_call(
        body,
        grid=(BGRID,),
        in_specs=[
            pl.BlockSpec((2, BROW, 8), lambda i: (0, i, 0)),
            pl.BlockSpec((BROW, 128), lambda i: (i, 0)),
            pl.BlockSpec((128, 128), lambda i: (0, 0)),
        ],
        out_specs=(
            pl.BlockSpec((BROW, 128), lambda i: (i, 0)),
            pl.BlockSpec((BROW, 1), lambda i: (i, 0)),
            pl.BlockSpec((BROW, FH), lambda i: (i, 0)),
            pl.BlockSpec((2, BROW, FH), lambda i: (0, i, 0)),
            pl.BlockSpec((2, BROW, FH), lambda i: (0, i, 0)),
            pl.BlockSpec((2, BROW, FH), lambda i: (0, i, 0)),
        ),
        out_shape=(jax.ShapeDtypeStruct((N, 128), jnp.float32),
                   jax.ShapeDtypeStruct((N, 1), jnp.float32),
                   jax.ShapeDtypeStruct((N, FH), jnp.float32),
                   jax.ShapeDtypeStruct((NCORE, N, FH), jnp.float32),
                   jax.ShapeDtypeStruct((NCORE, N, FH), jnp.float32),
                   jax.ShapeDtypeStruct((NCORE, N, FH), jnp.float32)),
    )(degp, x, W1p)


def _tc_layer(raw, P, dis, b, Wnext, F, Fn):
    fh = F // 2
    Fp, Fo = Wnext.shape

    def body(a_ref, p_ref, d_ref, b_ref, w_ref,
             pn_ref, u0_ref, qa_ref, qb_ref):
        dis = d_ref[...]
        t = jnp.concatenate(
            [p_ref[:, 0:fh] + dis * a_ref[0, :, 0:fh],
             p_ref[:, fh:F] + dis * a_ref[1, :, 0:fh]], axis=1)
        h = jnp.maximum(t + b_ref[...], 0.0)
        pn = jnp.dot(h, w_ref[...], preferred_element_type=jnp.float32,
                     precision=_PREC)
        pn_ref[...] = pn
        u0_ref[...] = _split16(dis, pn, Fn, 3, BROW)
        qa_ref[...] = _split16(dis, pn, Fn, 2, BROW)
        qb_ref[...] = _split16(dis, pn, Fn, 1, BROW)

    Fin = P.shape[1]
    return pl.pallas_call(
        body,
        grid=(BGRID,),
        in_specs=[
            pl.BlockSpec((2, BROW, FH), lambda i: (0, i, 0)),
            pl.BlockSpec((BROW, Fin), lambda i: (i, 0)),
            pl.BlockSpec((BROW, 1), lambda i: (i, 0)),
            pl.BlockSpec((1, Fp), lambda i: (0, 0)),
            pl.BlockSpec((Fp, Fo), lambda i: (0, 0)),
        ],
        out_specs=(
            pl.BlockSpec((BROW, Fo), lambda i: (i, 0)),
            pl.BlockSpec((2, BROW, FH), lambda i: (0, i, 0)),
            pl.BlockSpec((2, BROW, FH), lambda i: (0, i, 0)),
            pl.BlockSpec((2, BROW, FH), lambda i: (0, i, 0)),
        ),
        out_shape=(jax.ShapeDtypeStruct((N, Fo), jnp.float32),
                   jax.ShapeDtypeStruct((NCORE, N, FH), jnp.float32),
                   jax.ShapeDtypeStruct((NCORE, N, FH), jnp.float32),
                   jax.ShapeDtypeStruct((NCORE, N, FH), jnp.float32)),
    )(raw, P, dis, b.reshape(1, -1), Wnext)


def _tc_layer23(raw, P2, dis, b2, W3p):
    def body(a_ref, p_ref, d_ref, b_ref, w_ref,
             pn_ref, u0_ref, qa_ref, qb_ref):
        dis = d_ref[...]
        t = jnp.concatenate(
            [p_ref[:, 0:8] + dis * a_ref[0, :, 0:8],
             p_ref[:, 8:16] + dis * a_ref[1, :, 0:8]], axis=1)
        h = jnp.maximum(t + b_ref[...], 0.0)
        pn = jnp.dot(h, w_ref[...], preferred_element_type=jnp.float32,
                     precision=_PREC)
        pn_ref[...] = pn

        def dup(k):
            blk = jnp.concatenate(
                [dis * pn[:, 2 * k:2 * k + 2],
                 jnp.zeros((BROW, FH - 2), jnp.float32)], axis=1)
            return jnp.stack([blk, blk])

        u0_ref[...] = dup(3)
        qa_ref[...] = dup(2)
        qb_ref[...] = dup(1)

    return pl.pallas_call(
        body,
        grid=(BGRID,),
        in_specs=[
            pl.BlockSpec((2, BROW, FH), lambda i: (0, i, 0)),
            pl.BlockSpec((BROW, 64), lambda i: (i, 0)),
            pl.BlockSpec((BROW, 1), lambda i: (i, 0)),
            pl.BlockSpec((1, 16), lambda i: (0, 0)),
            pl.BlockSpec((16, 8), lambda i: (0, 0)),
        ],
        out_specs=(
            pl.BlockSpec((BROW, 8), lambda i: (i, 0)),
            pl.BlockSpec((2, BROW, FH), lambda i: (0, i, 0)),
            pl.BlockSpec((2, BROW, FH), lambda i: (0, i, 0)),
            pl.BlockSpec((2, BROW, FH), lambda i: (0, i, 0)),
        ),
        out_shape=(jax.ShapeDtypeStruct((N, 8), jnp.float32),
                   jax.ShapeDtypeStruct((NCORE, N, FH), jnp.float32),
                   jax.ShapeDtypeStruct((NCORE, N, FH), jnp.float32),
                   jax.ShapeDtypeStruct((NCORE, N, FH), jnp.float32)),
    )(raw, P2, dis, b2.reshape(1, -1), W3p)


def _tc_final(raw, P3, dis, b3):
    def body(a_ref, p_ref, d_ref, b_ref, o_ref):
        z = (p_ref[:, 0:2] + d_ref[...] * a_ref[0, :N, 0:2] + b_ref[...])
        m = jnp.max(z, axis=1, keepdims=True)
        lse = m + jnp.log(jnp.sum(jnp.exp(z - m), axis=1, keepdims=True))
        o_ref[...] = z - lse

    return pl.pallas_call(
        body, out_shape=jax.ShapeDtypeStruct((N, 2), jnp.float32),
    )(raw, P3, dis, b3.reshape(1, -1))


def kernel(x, edge_index, W1, b1, W2, b2, W3, b3):
    src = edge_index[0].astype(jnp.int32)
    dst = edge_index[1].astype(jnp.int32)
    pad = E_PAD - E
    src3 = jnp.concatenate([src, jnp.zeros((pad,), jnp.int32)]).reshape(
        NSUB, CH, CHUNK)
    dst3 = jnp.concatenate([dst, jnp.full((pad,), N, jnp.int32)]).reshape(
        NSUB, CH, CHUNK)
    src3d = src3.reshape(NSUB * 2, CH // 2, CHUNK)
    dst3d = dst3.reshape(NSUB * 2, CH // 2, CHUNK)
    del src3d

    ones8 = jnp.ones((CHUNK, 8), jnp.float32)
    z8 = jnp.zeros((ZROWS, 8), jnp.float32)

    W1p = jnp.concatenate([W1[i * 128:(i + 1) * 128] for i in range(4)], axis=1)
    W2p = jnp.concatenate([W2[i * 32:(i + 1) * 32] for i in range(4)], axis=1)
    W3p = jnp.concatenate([W3[i * 16:(i + 1) * 16] for i in range(4)], axis=1)

    degp = _deg_partials(dst3d, ones8, z8)
    P1, dis, d2, u0, qa, qb = _tc_pre(degp, x, W1p)

    raw1 = _layer_sc(u0, qa, qb, d2, src3, dst3)
    P2, u0, qa, qb = _tc_layer(raw1, P1, dis, b1, W2p, 32, 16)

    raw2 = _layer_sc(u0, qa, qb, d2, src3, dst3)
    P3, u0, qa, qb = _tc_layer23(raw2, P2, dis, b2, W3p)

    raw3 = _layer_sc(u0, qa, qb, d2, src3, dst3)
    return _tc_final(raw3, P3, dis, b3)


# speedup vs baseline: 1.2510x; 1.0003x over previous
import functools

import jax
import jax.numpy as jnp
from jax import lax
from jax.experimental import pallas as pl
from jax.experimental.pallas import tpu as pltpu
from jax.experimental.pallas import tpu_sc as plsc

N = 10000
E = 320000
NSUB = 16
NCORE = 2
CHUNK = 128
EPS = 20480
E_PAD = NSUB * EPS
CH = EPS // CHUNK
FH = 16
N_ACC = 10240
ZROWS = N_ACC // NSUB
CROWS = N // NSUB
CSPLIT = (128, 128, 128, 128, 113)
NBUF = 10
NDEPTH = 8

_PREC = jax.lax.Precision.HIGHEST


def _mesh():
    return plsc.VectorSubcoreMesh(core_axis_name="c", subcore_axis_name="s")


_SC_PARAMS = pltpu.CompilerParams(use_tc_tiling_on_sc=False)


def _deg_partials(dst3, ones_rows, zrows):
    @functools.partial(
        pl.kernel,
        out_type=jax.ShapeDtypeStruct((NCORE, N_ACC, 8), jnp.float32),
        mesh=_mesh(),
        scratch_types=[
            pltpu.VMEM((CH // 2, CHUNK), jnp.int32),
            pltpu.VMEM((CHUNK, 8), jnp.float32),
            pltpu.VMEM_SHARED((N_ACC, 8), jnp.float32),
            pltpu.SemaphoreType.DMA,
        ],
        compiler_params=_SC_PARAMS,
    )
    def deg_kernel(dst_hbm, ones_hbm, z_hbm, out_hbm, dstv, onesv, acc, sem):
        c = lax.axis_index("c")
        s = lax.axis_index("s")
        w = c * NSUB + s
        pltpu.sync_copy(z_hbm, acc.at[pl.ds(s * ZROWS, ZROWS)])
        pltpu.sync_copy(dst_hbm.at[w], dstv)
        pltpu.sync_copy(ones_hbm, onesv)
        plsc.subcore_barrier()

        @pl.loop(0, CH // 2)
        def _(j):
            pltpu.sync_copy(onesv, acc.at[dstv.at[j]], add=True)

        plsc.subcore_barrier()
        pltpu.sync_copy(acc.at[pl.ds(s * ZROWS, ZROWS)],
                        out_hbm.at[c, pl.ds(s * ZROWS, ZROWS)])

    return deg_kernel(dst3, ones_rows, zrows)


def _layer_sc(u0, qa, qb, d2, src3, dst3):
    @functools.partial(
        pl.kernel,
        out_type=(jax.ShapeDtypeStruct((NCORE, N_ACC, FH), jnp.float32),
                  jax.ShapeDtypeStruct((NCORE, N, FH), jnp.float32),
                  jax.ShapeDtypeStruct((NCORE, N, FH), jnp.float32)),
        mesh=_mesh(),
        scratch_types=[
            pltpu.VMEM((CH, CHUNK), jnp.int32),
            pltpu.VMEM((CH, CHUNK), jnp.int32),
            pltpu.VMEM((NBUF, CHUNK, FH), jnp.float32),
            pltpu.VMEM((CHUNK, FH), jnp.float32),
            pltpu.VMEM((CHUNK, FH), jnp.float32),
            pltpu.VMEM((CHUNK, FH), jnp.float32),
            pltpu.VMEM((CHUNK, FH), jnp.float32),
            pltpu.VMEM_SHARED((N_ACC, FH), jnp.float32),
            pltpu.SemaphoreType.DMA((NBUF,)),
            pltpu.SemaphoreType.DMA((NBUF,)),
        ],
        compiler_params=_SC_PARAMS,
    )
    def layer_kernel(u0_hbm, qa_hbm, qb_hbm, d2_hbm, src_hbm, dst_hbm,
                     raw_hbm, u1_hbm, u2_hbm,
                     srcv, dstv, rbuf, abuf, qbuf, dbuf, zbuf,
                     acc, gsem, ssem):
        c = lax.axis_index("c")
        s = lax.axis_index("s")

        pltpu.async_copy(src_hbm.at[s], srcv, gsem.at[0])
        pltpu.async_copy(dst_hbm.at[s], dstv, gsem.at[1])

        @pl.loop(0, CHUNK)
        def _(i):
            zbuf[i, :] = jnp.zeros((FH,), jnp.float32)

        for z in range(ZROWS // CHUNK):
            pltpu.async_copy(zbuf, acc.at[pl.ds(s * ZROWS + z * CHUNK, CHUNK)],
                             ssem.at[z])
        pltpu.make_async_copy(src_hbm.at[s], srcv, gsem.at[0]).wait()
        pltpu.make_async_copy(dst_hbm.at[s], dstv, gsem.at[1]).wait()
        for z in range(ZROWS // CHUNK):
            pltpu.make_async_copy(
                zbuf, acc.at[pl.ds(s * ZROWS + z * CHUNK, CHUNK)],
                ssem.at[z]).wait()
        plsc.subcore_barrier()

        def hop(u_hbm):
            usrc = u_hbm.at[c]
            for kk in range(NDEPTH):
                pltpu.async_copy(usrc.at[srcv.at[kk]], rbuf.at[kk],
                                 gsem.at[kk])

            @pl.loop(0, CH, step=NBUF)
            def _(j):
                for l in range(NBUF):
                    k = j + l
                    pltpu.make_async_copy(usrc.at[srcv.at[k]], rbuf.at[l],
                                          gsem.at[l]).wait()
                    pltpu.async_copy(rbuf.at[l], acc.at[dstv.at[k]],
                                     ssem.at[l], add=True)
                    la = (l + NDEPTH) % NBUF
                    ka = k + NDEPTH

                    @pl.when(ka < CH)
                    def _():
                        @pl.when(ka >= NBUF)
                        def _():
                            pltpu.make_async_copy(
                                rbuf.at[la], acc.at[dstv.at[k]],
                                ssem.at[la]).wait()

                        pltpu.async_copy(usrc.at[srcv.at[ka]], rbuf.at[la],
                                         gsem.at[la])

            for l in range(NBUF):
                pltpu.make_async_copy(rbuf.at[l],
                                      acc.at[dstv.at[CH - NBUF + l]],
                                      ssem.at[l]).wait()
            plsc.subcore_barrier()

        def combine(q_hbm, unext_hbm):
            off = s * CROWS
            for sz in CSPLIT:
                pltpu.sync_copy(acc.at[pl.ds(off, sz)], abuf.at[pl.ds(0, sz)])
                pltpu.sync_copy(q_hbm.at[c, pl.ds(off, sz)],
                                qbuf.at[pl.ds(0, sz)])
                pltpu.sync_copy(d2_hbm.at[pl.ds(off, sz)],
                                dbuf.at[pl.ds(0, sz)])

                @pl.loop(0, sz)
                def _(i):
                    abuf[i, :] = qbuf[i, :] + dbuf[i, :] * abuf[i, :]

                pltpu.sync_copy(abuf.at[pl.ds(0, sz)],
                                unext_hbm.at[c, pl.ds(off, sz)])
                pltpu.sync_copy(zbuf.at[pl.ds(0, sz)], acc.at[pl.ds(off, sz)])
                off += sz
            plsc.subcore_barrier()

        hop(u0_hbm)
        combine(qa_hbm, u1_hbm)
        hop(u1_hbm)
        combine(qb_hbm, u2_hbm)
        hop(u2_hbm)

        for z in range(ZROWS // CHUNK):
            sl = pl.ds(s * ZROWS + z * CHUNK, CHUNK)
            pltpu.sync_copy(acc.at[sl], raw_hbm.at[c, sl])

    return layer_kernel(u0, qa, qb, d2, src3, dst3)[0]


BROW = 2000
BGRID = N // BROW


def _split16(dis, P, F, k, rows):
    fh = F // 2
    blocks = []
    for c in range(2):
        blk = dis * P[:, k * F + c * fh:k * F + (c + 1) * fh]
        if fh < FH:
            blk = jnp.concatenate(
                [blk, jnp.zeros((rows, FH - fh), jnp.float32)], axis=1)
        blocks.append(blk)
    return jnp.stack(blocks)


def _tc_pre(degp, x, W1p):
    def body(d_ref, x_ref, w_ref, p_ref, dis_ref, d2_ref,
             u0_ref, qa_ref, qb_ref):
        deg = d_ref[0, :, 0:1] + d_ref[1, :, 0:1]
        dis = jnp.where(deg > 0.0,
                        lax.rsqrt(jnp.maximum(deg, 1e-12)),
                        0.0)
        dis_ref[...] = dis
        d2_ref[...] = jnp.broadcast_to(dis * dis, (BROW, FH))
        P1 = jnp.dot(x_ref[...], w_ref[...],
                     preferred_element_type=jnp.float32, precision=_PREC)
        p_ref[...] = P1
        u0_ref[...] = _split16(dis, P1, 32, 3, BROW)
        qa_ref[...] = _split16(dis, P1, 32, 2, BROW)
        qb_ref[...] = _split16(dis, P1, 32, 1, BROW)

    return pl.pallas_call(
        body,
        grid=(BGRID,),
        in_specs=[
            pl.BlockSpec((2, BROW, 8), lambda i: (0, i, 0)),
            pl.BlockSpec((BROW, 128), lambda i: (i, 0)),
            pl.BlockSpec((128, 128), lambda i: (0, 0)),
        ],
        out_specs=(
            pl.BlockSpec((BROW, 128), lambda i: (i, 0)),
            pl.BlockSpec((BROW, 1), lambda i: (i, 0)),
            pl.BlockSpec((BROW, FH), lambda i: (i, 0)),
            pl.BlockSpec((2, BROW, FH), lambda i: (0, i, 0)),
            pl.BlockSpec((2, BROW, FH), lambda i: (0, i, 0)),
            pl.BlockSpec((2, BROW, FH), lambda i: (0, i, 0)),
        ),
        out_shape=(jax.ShapeDtypeStruct((N, 128), jnp.float32),
                   jax.ShapeDtypeStruct((N, 1), jnp.float32),
                   jax.ShapeDtypeStruct((N, FH), jnp.float32),
                   jax.ShapeDtypeStruct((NCORE, N, FH), jnp.float32),
                   jax.ShapeDtypeStruct((NCORE, N, FH), jnp.float32),
                   jax.ShapeDtypeStruct((NCORE, N, FH), jnp.float32)),
    )(degp, x, W1p)


def _tc_layer(raw, P, dis, b, Wnext, F, Fn):
    fh = F // 2
    Fp, Fo = Wnext.shape

    def body(a_ref, p_ref, d_ref, b_ref, w_ref,
             pn_ref, u0_ref, qa_ref, qb_ref):
        dis = d_ref[...]
        t = jnp.concatenate(
            [p_ref[:, 0:fh] + dis * a_ref[0, :, 0:fh],
             p_ref[:, fh:F] + dis * a_ref[1, :, 0:fh]], axis=1)
        h = jnp.maximum(t + b_ref[...], 0.0)
        pn = jnp.dot(h, w_ref[...], preferred_element_type=jnp.float32,
                     precision=_PREC)
        pn_ref[...] = pn
        u0_ref[...] = _split16(dis, pn, Fn, 3, BROW)
        qa_ref[...] = _split16(dis, pn, Fn, 2, BROW)
        qb_ref[...] = _split16(dis, pn, Fn, 1, BROW)

    Fin = P.shape[1]
    return pl.pallas_call(
        body,
        grid=(BGRID,),
        in_specs=[
            pl.BlockSpec((2, BROW, FH), lambda i: (0, i, 0)),
            pl.BlockSpec((BROW, Fin), lambda i: (i, 0)),
            pl.BlockSpec((BROW, 1), lambda i: (i, 0)),
            pl.BlockSpec((1, Fp), lambda i: (0, 0)),
            pl.BlockSpec((Fp, Fo), lambda i: (0, 0)),
        ],
        out_specs=(
            pl.BlockSpec((BROW, Fo), lambda i: (i, 0)),
            pl.BlockSpec((2, BROW, FH), lambda i: (0, i, 0)),
            pl.BlockSpec((2, BROW, FH), lambda i: (0, i, 0)),
            pl.BlockSpec((2, BROW, FH), lambda i: (0, i, 0)),
        ),
        out_shape=(jax.ShapeDtypeStruct((N, Fo), jnp.float32),
                   jax.ShapeDtypeStruct((NCORE, N, FH), jnp.float32),
                   jax.ShapeDtypeStruct((NCORE, N, FH), jnp.float32),
                   jax.ShapeDtypeStruct((NCORE, N, FH), jnp.float32)),
    )(raw, P, dis, b.reshape(1, -1), Wnext)


def _tc_layer23(raw, P2, dis, b2, W3p):
    def body(a_ref, p_ref, d_ref, b_ref, w_ref,
             pn_ref, u0_ref, qa_ref, qb_ref):
        dis = d_ref[...]
        t = jnp.concatenate(
            [p_ref[:, 0:8] + dis * a_ref[0, :, 0:8],
             p_ref[:, 8:16] + dis * a_ref[1, :, 0:8]], axis=1)
        h = jnp.maximum(t + b_ref[...], 0.0)
        pn = jnp.dot(h, w_ref[...], preferred_element_type=jnp.float32,
                     precision=_PREC)
        pn_ref[...] = pn

        def dup(k):
            blk = jnp.concatenate(
                [dis * pn[:, 2 * k:2 * k + 2],
                 jnp.zeros((BROW, FH - 2), jnp.float32)], axis=1)
            return jnp.stack([blk, blk])

        u0_ref[...] = dup(3)
        qa_ref[...] = dup(2)
        qb_ref[...] = dup(1)

    return pl.pallas_call(
        body,
        grid=(BGRID,),
        in_specs=[
            pl.BlockSpec((2, BROW, FH), lambda i: (0, i, 0)),
            pl.BlockSpec((BROW, 64), lambda i: (i, 0)),
            pl.BlockSpec((BROW, 1), lambda i: (i, 0)),
            pl.BlockSpec((1, 16), lambda i: (0, 0)),
            pl.BlockSpec((16, 8), lambda i: (0, 0)),
        ],
        out_specs=(
            pl.BlockSpec((BROW, 8), lambda i: (i, 0)),
            pl.BlockSpec((2, BROW, FH), lambda i: (0, i, 0)),
            pl.BlockSpec((2, BROW, FH), lambda i: (0, i, 0)),
            pl.BlockSpec((2, BROW, FH), lambda i: (0, i, 0)),
        ),
        out_shape=(jax.ShapeDtypeStruct((N, 8), jnp.float32),
                   jax.ShapeDtypeStruct((NCORE, N, FH), jnp.float32),
                   jax.ShapeDtypeStruct((NCORE, N, FH), jnp.float32),
                   jax.ShapeDtypeStruct((NCORE, N, FH), jnp.float32)),
    )(raw, P2, dis, b2.reshape(1, -1), W3p)


def _tc_final(raw, P3, dis, b3):
    def body(a_ref, p_ref, d_ref, b_ref, o_ref):
        z = (p_ref[:, 0:2] + d_ref[...] * a_ref[0, :N, 0:2] + b_ref[...])
        m = jnp.max(z, axis=1, keepdims=True)
        lse = m + jnp.log(jnp.sum(jnp.exp(z - m), axis=1, keepdims=True))
        o_ref[...] = z - lse

    return pl.pallas_call(
        body, out_shape=jax.ShapeDtypeStruct((N, 2), jnp.float32),
    )(raw, P3, dis, b3.reshape(1, -1))


def kernel(x, edge_index, W1, b1, W2, b2, W3, b3):
    src = edge_index[0].astype(jnp.int32)
    dst = edge_index[1].astype(jnp.int32)
    pad = E_PAD - E
    src3 = jnp.concatenate([src, jnp.zeros((pad,), jnp.int32)]).reshape(
        NSUB, CH, CHUNK)
    dst3 = jnp.concatenate([dst, jnp.full((pad,), N, jnp.int32)]).reshape(
        NSUB, CH, CHUNK)
    src3d = src3.reshape(NSUB * 2, CH // 2, CHUNK)
    dst3d = dst3.reshape(NSUB * 2, CH // 2, CHUNK)
    del src3d

    ones8 = jnp.ones((CHUNK, 8), jnp.float32)
    z8 = jnp.zeros((ZROWS, 8), jnp.float32)

    W1p = jnp.concatenate([W1[i * 128:(i + 1) * 128] for i in range(4)], axis=1)
    W2p = jnp.concatenate([W2[i * 32:(i + 1) * 32] for i in range(4)], axis=1)
    W3p = jnp.concatenate([W3[i * 16:(i + 1) * 16] for i in range(4)], axis=1)

    degp = _deg_partials(dst3d, ones8, z8)
    P1, dis, d2, u0, qa, qb = _tc_pre(degp, x, W1p)

    raw1 = _layer_sc(u0, qa, qb, d2, src3, dst3)
    P2, u0, qa, qb = _tc_layer(raw1, P1, dis, b1, W2p, 32, 16)

    raw2 = _layer_sc(u0, qa, qb, d2, src3, dst3)
    P3, u0, qa, qb = _tc_layer23(raw2, P2, dis, b2, W3p)

    raw3 = _layer_sc(u0, qa, qb, d2, src3, dst3)
    return _tc_final(raw3, P3, dis, b3)


# speedup vs baseline: 1.2876x; 1.0293x over previous
import functools

import jax
import jax.numpy as jnp
from jax import lax
from jax.experimental import pallas as pl
from jax.experimental.pallas import tpu as pltpu
from jax.experimental.pallas import tpu_sc as plsc

N = 10000
E = 320000
NSUB = 16
NCORE = 2
CHUNK = 128
EPS = 20480
E_PAD = NSUB * EPS
CH = EPS // CHUNK
FH = 16
N_ACC = 10240
ZROWS = N_ACC // NSUB
CROWS = N // NSUB
CSPLIT = (128, 128, 128, 128, 113)
NBUF = 10
NDEPTH = 8

_PREC = jax.lax.Precision.HIGHEST


def _mesh():
    return plsc.VectorSubcoreMesh(core_axis_name="c", subcore_axis_name="s")


_SC_PARAMS = pltpu.CompilerParams(use_tc_tiling_on_sc=False)


def _deg_partials(dst3, ones_rows, zrows):
    @functools.partial(
        pl.kernel,
        out_type=jax.ShapeDtypeStruct((NCORE, N_ACC, 8), jnp.float32),
        mesh=_mesh(),
        scratch_types=[
            pltpu.VMEM((CH // 2, CHUNK), jnp.int32),
            pltpu.VMEM((CHUNK, 8), jnp.float32),
            pltpu.VMEM_SHARED((N_ACC, 8), jnp.float32),
            pltpu.SemaphoreType.DMA,
        ],
        compiler_params=_SC_PARAMS,
    )
    def deg_kernel(dst_hbm, ones_hbm, z_hbm, out_hbm, dstv, onesv, acc, sem):
        c = lax.axis_index("c")
        s = lax.axis_index("s")
        w = c * NSUB + s
        pltpu.sync_copy(z_hbm, acc.at[pl.ds(s * ZROWS, ZROWS)])
        pltpu.sync_copy(dst_hbm.at[w], dstv)
        pltpu.sync_copy(ones_hbm, onesv)
        plsc.subcore_barrier()

        @pl.loop(0, CH // 2)
        def _(j):
            pltpu.sync_copy(onesv, acc.at[dstv.at[j]], add=True)

        plsc.subcore_barrier()
        pltpu.sync_copy(acc.at[pl.ds(s * ZROWS, ZROWS)],
                        out_hbm.at[c, pl.ds(s * ZROWS, ZROWS)])

    return deg_kernel(dst3, ones_rows, zrows)


def _layer_sc(u0, qa, qb, d2, src3, dst3):
    @functools.partial(
        pl.kernel,
        out_type=(jax.ShapeDtypeStruct((NCORE, N_ACC, FH), jnp.float32),
                  jax.ShapeDtypeStruct((NCORE, N, FH), jnp.float32),
                  jax.ShapeDtypeStruct((NCORE, N, FH), jnp.float32)),
        mesh=_mesh(),
        scratch_types=[
            pltpu.VMEM((CH, CHUNK), jnp.int32),
            pltpu.VMEM((CH, CHUNK), jnp.int32),
            pltpu.VMEM((NBUF, CHUNK, FH), jnp.float32),
            pltpu.VMEM((2, CHUNK, FH), jnp.float32),
            pltpu.VMEM((CHUNK, FH), jnp.float32),
            pltpu.VMEM((CHUNK, FH), jnp.float32),
            pltpu.VMEM((CHUNK, FH), jnp.float32),
            pltpu.VMEM_SHARED((N_ACC, FH), jnp.float32),
            pltpu.SemaphoreType.DMA((NBUF,)),
            pltpu.SemaphoreType.DMA((NBUF,)),
        ],
        compiler_params=_SC_PARAMS,
    )
    def layer_kernel(u0_hbm, qa_hbm, qb_hbm, d2_hbm, src_hbm, dst_hbm,
                     raw_hbm, u1_hbm, u2_hbm,
                     srcv, dstv, rbuf, abuf, qbuf, dbuf, zbuf,
                     acc, gsem, ssem):
        c = lax.axis_index("c")
        s = lax.axis_index("s")

        pltpu.async_copy(src_hbm.at[s], srcv, gsem.at[0])
        pltpu.async_copy(dst_hbm.at[s], dstv, gsem.at[1])

        @pl.loop(0, CHUNK)
        def _(i):
            zbuf[i, :] = jnp.zeros((FH,), jnp.float32)

        for z in range(ZROWS // CHUNK):
            pltpu.async_copy(zbuf, acc.at[pl.ds(s * ZROWS + z * CHUNK, CHUNK)],
                             ssem.at[z])
        pltpu.make_async_copy(src_hbm.at[s], srcv, gsem.at[0]).wait()
        pltpu.make_async_copy(dst_hbm.at[s], dstv, gsem.at[1]).wait()
        for z in range(ZROWS // CHUNK):
            pltpu.make_async_copy(
                zbuf, acc.at[pl.ds(s * ZROWS + z * CHUNK, CHUNK)],
                ssem.at[z]).wait()
        plsc.subcore_barrier()

        def hop(u_hbm):
            usrc = u_hbm.at[c]
            for kk in range(NDEPTH):
                pltpu.async_copy(usrc.at[srcv.at[kk]], rbuf.at[kk],
                                 gsem.at[kk])

            @pl.loop(0, CH, step=NBUF)
            def _(j):
                for l in range(NBUF):
                    k = j + l
                    pltpu.make_async_copy(usrc.at[srcv.at[k]], rbuf.at[l],
                                          gsem.at[l]).wait()
                    pltpu.async_copy(rbuf.at[l], acc.at[dstv.at[k]],
                                     ssem.at[l], add=True)
                    la = (l + NDEPTH) % NBUF
                    ka = k + NDEPTH

                    @pl.when(ka < CH)
                    def _():
                        @pl.when(ka >= NBUF)
                        def _():
                            pltpu.make_async_copy(
                                rbuf.at[la], acc.at[dstv.at[k]],
                                ssem.at[la]).wait()

                        pltpu.async_copy(usrc.at[srcv.at[ka]], rbuf.at[la],
                                         gsem.at[la])

            for l in range(NBUF):
                pltpu.make_async_copy(rbuf.at[l],
                                      acc.at[dstv.at[CH - NBUF + l]],
                                      ssem.at[l]).wait()
            plsc.subcore_barrier()

        def combine(q_hbm, unext_hbm):
            offs = []
            off = s * CROWS
            for sz in CSPLIT:
                offs.append((off, sz))
                off += sz
            for idx, (off, sz) in enumerate(offs):
                ab = abuf.at[idx % 2]
                if idx >= 2:
                    poff, psz = offs[idx - 2]
                    pltpu.make_async_copy(ab.at[pl.ds(0, psz)],
                                          unext_hbm.at[c, pl.ds(poff, psz)],
                                          ssem.at[idx - 2]).wait()
                pltpu.async_copy(acc.at[pl.ds(off, sz)], ab.at[pl.ds(0, sz)],
                                 gsem.at[0])
                pltpu.async_copy(q_hbm.at[c, pl.ds(off, sz)],
                                 qbuf.at[pl.ds(0, sz)], gsem.at[1])
                pltpu.async_copy(d2_hbm.at[pl.ds(off, sz)],
                                 dbuf.at[pl.ds(0, sz)], gsem.at[2])
                pltpu.make_async_copy(acc.at[pl.ds(off, sz)],
                                      ab.at[pl.ds(0, sz)], gsem.at[0]).wait()
                pltpu.make_async_copy(q_hbm.at[c, pl.ds(off, sz)],
                                      qbuf.at[pl.ds(0, sz)], gsem.at[1]).wait()
                pltpu.make_async_copy(d2_hbm.at[pl.ds(off, sz)],
                                      dbuf.at[pl.ds(0, sz)], gsem.at[2]).wait()

                @pl.loop(0, sz)
                def _(i):
                    ab[i, :] = qbuf[i, :] + dbuf[i, :] * ab[i, :]

                pltpu.async_copy(ab.at[pl.ds(0, sz)],
                                 unext_hbm.at[c, pl.ds(off, sz)], ssem.at[idx])
                pltpu.async_copy(zbuf.at[pl.ds(0, sz)], acc.at[pl.ds(off, sz)],
                                 ssem.at[5 + idx])
            for idx, (off, sz) in enumerate(offs):
                if idx >= len(offs) - 2:
                    pltpu.make_async_copy(abuf.at[idx % 2].at[pl.ds(0, sz)],
                                          unext_hbm.at[c, pl.ds(off, sz)],
                                          ssem.at[idx]).wait()
                pltpu.make_async_copy(zbuf.at[pl.ds(0, sz)],
                                      acc.at[pl.ds(off, sz)],
                                      ssem.at[5 + idx]).wait()
            plsc.subcore_barrier()

        hop(u0_hbm)
        combine(qa_hbm, u1_hbm)
        hop(u1_hbm)
        combine(qb_hbm, u2_hbm)
        hop(u2_hbm)

        for z in range(ZROWS // CHUNK):
            sl = pl.ds(s * ZROWS + z * CHUNK, CHUNK)
            pltpu.sync_copy(acc.at[sl], raw_hbm.at[c, sl])

    return layer_kernel(u0, qa, qb, d2, src3, dst3)[0]


BROW = 2000
BGRID = N // BROW


def _split16(dis, P, F, k, rows):
    fh = F // 2
    blocks = []
    for c in range(2):
        blk = dis * P[:, k * F + c * fh:k * F + (c + 1) * fh]
        if fh < FH:
            blk = jnp.concatenate(
                [blk, jnp.zeros((rows, FH - fh), jnp.float32)], axis=1)
        blocks.append(blk)
    return jnp.stack(blocks)


def _tc_pre(degp, x, W1p):
    def body(d_ref, x_ref, w_ref, p_ref, dis_ref, d2_ref,
             u0_ref, qa_ref, qb_ref):
        deg = d_ref[0, :, 0:1] + d_ref[1, :, 0:1]
        dis = jnp.where(deg > 0.0,
                        lax.rsqrt(jnp.maximum(deg, 1e-12)),
                        0.0)
        dis_ref[...] = dis
        d2_ref[...] = jnp.broadcast_to(dis * dis, (BROW, FH))
        P1 = jnp.dot(x_ref[...], w_ref[...],
                     preferred_element_type=jnp.float32, precision=_PREC)
        p_ref[...] = P1
        u0_ref[...] = _split16(dis, P1, 32, 3, BROW)
        qa_ref[...] = _split16(dis, P1, 32, 2, BROW)
        qb_ref[...] = _split16(dis, P1, 32, 1, BROW)

    return pl.pallas_call(
        body,
        grid=(BGRID,),
        in_specs=[
            pl.BlockSpec((2, BROW, 8), lambda i: (0, i, 0)),
            pl.BlockSpec((BROW, 128), lambda i: (i, 0)),
            pl.BlockSpec((128, 128), lambda i: (0, 0)),
        ],
        out_specs=(
            pl.BlockSpec((BROW, 128), lambda i: (i, 0)),
            pl.BlockSpec((BROW, 1), lambda i: (i, 0)),
            pl.BlockSpec((BROW, FH), lambda i: (i, 0)),
            pl.BlockSpec((2, BROW, FH), lambda i: (0, i, 0)),
            pl.BlockSpec((2, BROW, FH), lambda i: (0, i, 0)),
            pl.BlockSpec((2, BROW, FH), lambda i: (0, i, 0)),
        ),
        out_shape=(jax.ShapeDtypeStruct((N, 128), jnp.float32),
                   jax.ShapeDtypeStruct((N, 1), jnp.float32),
                   jax.ShapeDtypeStruct((N, FH), jnp.float32),
                   jax.ShapeDtypeStruct((NCORE, N, FH), jnp.float32),
                   jax.ShapeDtypeStruct((NCORE, N, FH), jnp.float32),
                   jax.ShapeDtypeStruct((NCORE, N, FH), jnp.float32)),
    )(degp, x, W1p)


def _tc_layer(raw, P, dis, b, Wnext, F, Fn):
    fh = F // 2
    Fp, Fo = Wnext.shape

    def body(a_ref, p_ref, d_ref, b_ref, w_ref,
             pn_ref, u0_ref, qa_ref, qb_ref):
        dis = d_ref[...]
        t = jnp.concatenate(
            [p_ref[:, 0:fh] + dis * a_ref[0, :, 0:fh],
             p_ref[:, fh:F] + dis * a_ref[1, :, 0:fh]], axis=1)
        h = jnp.maximum(t + b_ref[...], 0.0)
        pn = jnp.dot(h, w_ref[...], preferred_element_type=jnp.float32,
                     precision=_PREC)
        pn_ref[...] = pn
        u0_ref[...] = _split16(dis, pn, Fn, 3, BROW)
        qa_ref[...] = _split16(dis, pn, Fn, 2, BROW)
        qb_ref[...] = _split16(dis, pn, Fn, 1, BROW)

    Fin = P.shape[1]
    return pl.pallas_call(
        body,
        grid=(BGRID,),
        in_specs=[
            pl.BlockSpec((2, BROW, FH), lambda i: (0, i, 0)),
            pl.BlockSpec((BROW, Fin), lambda i: (i, 0)),
            pl.BlockSpec((BROW, 1), lambda i: (i, 0)),
            pl.BlockSpec((1, Fp), lambda i: (0, 0)),
            pl.BlockSpec((Fp, Fo), lambda i: (0, 0)),
        ],
        out_specs=(
            pl.BlockSpec((BROW, Fo), lambda i: (i, 0)),
            pl.BlockSpec((2, BROW, FH), lambda i: (0, i, 0)),
            pl.BlockSpec((2, BROW, FH), lambda i: (0, i, 0)),
            pl.BlockSpec((2, BROW, FH), lambda i: (0, i, 0)),
        ),
        out_shape=(jax.ShapeDtypeStruct((N, Fo), jnp.float32),
                   jax.ShapeDtypeStruct((NCORE, N, FH), jnp.float32),
                   jax.ShapeDtypeStruct((NCORE, N, FH), jnp.float32),
                   jax.ShapeDtypeStruct((NCORE, N, FH), jnp.float32)),
    )(raw, P, dis, b.reshape(1, -1), Wnext)


def _tc_layer23(raw, P2, dis, b2, W3p):
    def body(a_ref, p_ref, d_ref, b_ref, w_ref,
             pn_ref, u0_ref, qa_ref, qb_ref):
        dis = d_ref[...]
        t = jnp.concatenate(
            [p_ref[:, 0:8] + dis * a_ref[0, :, 0:8],
             p_ref[:, 8:16] + dis * a_ref[1, :, 0:8]], axis=1)
        h = jnp.maximum(t + b_ref[...], 0.0)
        pn = jnp.dot(h, w_ref[...], preferred_element_type=jnp.float32,
                     precision=_PREC)
        pn_ref[...] = pn

        def dup(k):
            blk = jnp.concatenate(
                [dis * pn[:, 2 * k:2 * k + 2],
                 jnp.zeros((BROW, FH - 2), jnp.float32)], axis=1)
            return jnp.stack([blk, blk])

        u0_ref[...] = dup(3)
        qa_ref[...] = dup(2)
        qb_ref[...] = dup(1)

    return pl.pallas_call(
        body,
        grid=(BGRID,),
        in_specs=[
            pl.BlockSpec((2, BROW, FH), lambda i: (0, i, 0)),
            pl.BlockSpec((BROW, 64), lambda i: (i, 0)),
            pl.BlockSpec((BROW, 1), lambda i: (i, 0)),
            pl.BlockSpec((1, 16), lambda i: (0, 0)),
            pl.BlockSpec((16, 8), lambda i: (0, 0)),
        ],
        out_specs=(
            pl.BlockSpec((BROW, 8), lambda i: (i, 0)),
            pl.BlockSpec((2, BROW, FH), lambda i: (0, i, 0)),
            pl.BlockSpec((2, BROW, FH), lambda i: (0, i, 0)),
            pl.BlockSpec((2, BROW, FH), lambda i: (0, i, 0)),
        ),
        out_shape=(jax.ShapeDtypeStruct((N, 8), jnp.float32),
                   jax.ShapeDtypeStruct((NCORE, N, FH), jnp.float32),
                   jax.ShapeDtypeStruct((NCORE, N, FH), jnp.float32),
                   jax.ShapeDtypeStruct((NCORE, N, FH), jnp.float32)),
    )(raw, P2, dis, b2.reshape(1, -1), W3p)


def _tc_final(raw, P3, dis, b3):
    def body(a_ref, p_ref, d_ref, b_ref, o_ref):
        z = (p_ref[:, 0:2] + d_ref[...] * a_ref[0, :N, 0:2] + b_ref[...])
        m = jnp.max(z, axis=1, keepdims=True)
        lse = m + jnp.log(jnp.sum(jnp.exp(z - m), axis=1, keepdims=True))
        o_ref[...] = z - lse

    return pl.pallas_call(
        body, out_shape=jax.ShapeDtypeStruct((N, 2), jnp.float32),
    )(raw, P3, dis, b3.reshape(1, -1))


def kernel(x, edge_index, W1, b1, W2, b2, W3, b3):
    src = edge_index[0].astype(jnp.int32)
    dst = edge_index[1].astype(jnp.int32)
    pad = E_PAD - E
    src3 = jnp.concatenate([src, jnp.zeros((pad,), jnp.int32)]).reshape(
        NSUB, CH, CHUNK)
    dst3 = jnp.concatenate([dst, jnp.full((pad,), N, jnp.int32)]).reshape(
        NSUB, CH, CHUNK)
    src3d = src3.reshape(NSUB * 2, CH // 2, CHUNK)
    dst3d = dst3.reshape(NSUB * 2, CH // 2, CHUNK)
    del src3d

    ones8 = jnp.ones((CHUNK, 8), jnp.float32)
    z8 = jnp.zeros((ZROWS, 8), jnp.float32)

    W1p = jnp.concatenate([W1[i * 128:(i + 1) * 128] for i in range(4)], axis=1)
    W2p = jnp.concatenate([W2[i * 32:(i + 1) * 32] for i in range(4)], axis=1)
    W3p = jnp.concatenate([W3[i * 16:(i + 1) * 16] for i in range(4)], axis=1)

    degp = _deg_partials(dst3d, ones8, z8)
    P1, dis, d2, u0, qa, qb = _tc_pre(degp, x, W1p)

    raw1 = _layer_sc(u0, qa, qb, d2, src3, dst3)
    P2, u0, qa, qb = _tc_layer(raw1, P1, dis, b1, W2p, 32, 16)

    raw2 = _layer_sc(u0, qa, qb, d2, src3, dst3)
    P3, u0, qa, qb = _tc_layer23(raw2, P2, dis, b2, W3p)

    raw3 = _layer_sc(u0, qa, qb, d2, src3, dst3)
    return _tc_final(raw3, P3, dis, b3)


# speedup vs baseline: 1.2970x; 1.0073x over previous
import functools

import jax
import jax.numpy as jnp
from jax import lax
from jax.experimental import pallas as pl
from jax.experimental.pallas import tpu as pltpu
from jax.experimental.pallas import tpu_sc as plsc

N = 10000
E = 320000
NSUB = 16
NCORE = 2
CHUNK = 128
EPS = 20480
E_PAD = NSUB * EPS
CH = EPS // CHUNK
FH = 16
N_ACC = 10240
ZROWS = N_ACC // NSUB
CROWS = N // NSUB
CSPLIT = (128, 128, 128, 128, 113)
NBUF = 10
NDEPTH = 8

_PREC = jax.lax.Precision.HIGHEST


def _mesh():
    return plsc.VectorSubcoreMesh(core_axis_name="c", subcore_axis_name="s")


_SC_PARAMS = pltpu.CompilerParams(use_tc_tiling_on_sc=False)


def _deg_partials(dst3, ones_rows, zrows):
    @functools.partial(
        pl.kernel,
        out_type=jax.ShapeDtypeStruct((NCORE, N_ACC, 8), jnp.float32),
        mesh=_mesh(),
        scratch_types=[
            pltpu.VMEM((CH // 2, CHUNK), jnp.int32),
            pltpu.VMEM((CHUNK, 8), jnp.float32),
            pltpu.VMEM_SHARED((N_ACC, 8), jnp.float32),
            pltpu.SemaphoreType.DMA((4,)),
        ],
        compiler_params=_SC_PARAMS,
    )
    def deg_kernel(dst_hbm, ones_hbm, z_hbm, out_hbm, dstv, onesv, acc, sem):
        c = lax.axis_index("c")
        s = lax.axis_index("s")
        w = c * NSUB + s
        pltpu.async_copy(z_hbm, acc.at[pl.ds(s * ZROWS, ZROWS)], sem.at[0])
        pltpu.async_copy(dst_hbm.at[w], dstv, sem.at[1])
        pltpu.async_copy(ones_hbm, onesv, sem.at[2])
        pltpu.make_async_copy(z_hbm, acc.at[pl.ds(s * ZROWS, ZROWS)],
                              sem.at[0]).wait()
        pltpu.make_async_copy(dst_hbm.at[w], dstv, sem.at[1]).wait()
        pltpu.make_async_copy(ones_hbm, onesv, sem.at[2]).wait()
        plsc.subcore_barrier()

        @pl.loop(0, CH // 2, step=4)
        def _(j):
            for l in range(4):
                pltpu.async_copy(onesv, acc.at[dstv.at[j + l]], sem.at[l],
                                 add=True)
            for l in range(4):
                pltpu.make_async_copy(onesv, acc.at[dstv.at[j + l]],
                                      sem.at[l]).wait()

        plsc.subcore_barrier()
        pltpu.sync_copy(acc.at[pl.ds(s * ZROWS, ZROWS)],
                        out_hbm.at[c, pl.ds(s * ZROWS, ZROWS)])

    return deg_kernel(dst3, ones_rows, zrows)


def _layer_sc(u0, qa, qb, d2, src3, dst3):
    @functools.partial(
        pl.kernel,
        out_type=(jax.ShapeDtypeStruct((NCORE, N_ACC, FH), jnp.float32),
                  jax.ShapeDtypeStruct((NCORE, N, FH), jnp.float32),
                  jax.ShapeDtypeStruct((NCORE, N, FH), jnp.float32)),
        mesh=_mesh(),
        scratch_types=[
            pltpu.VMEM((CH, CHUNK), jnp.int32),
            pltpu.VMEM((CH, CHUNK), jnp.int32),
            pltpu.VMEM((NBUF, CHUNK, FH), jnp.float32),
            pltpu.VMEM((2, CHUNK, FH), jnp.float32),
            pltpu.VMEM((CHUNK, FH), jnp.float32),
            pltpu.VMEM((CHUNK, FH), jnp.float32),
            pltpu.VMEM((CHUNK, FH), jnp.float32),
            pltpu.VMEM_SHARED((N_ACC, FH), jnp.float32),
            pltpu.SemaphoreType.DMA((NBUF,)),
            pltpu.SemaphoreType.DMA((NBUF,)),
        ],
        compiler_params=_SC_PARAMS,
    )
    def layer_kernel(u0_hbm, qa_hbm, qb_hbm, d2_hbm, src_hbm, dst_hbm,
                     raw_hbm, u1_hbm, u2_hbm,
                     srcv, dstv, rbuf, abuf, qbuf, dbuf, zbuf,
                     acc, gsem, ssem):
        c = lax.axis_index("c")
        s = lax.axis_index("s")

        pltpu.async_copy(src_hbm.at[s], srcv, gsem.at[0])
        pltpu.async_copy(dst_hbm.at[s], dstv, gsem.at[1])

        @pl.loop(0, CHUNK)
        def _(i):
            zbuf[i, :] = jnp.zeros((FH,), jnp.float32)

        for z in range(ZROWS // CHUNK):
            pltpu.async_copy(zbuf, acc.at[pl.ds(s * ZROWS + z * CHUNK, CHUNK)],
                             ssem.at[z])
        pltpu.make_async_copy(src_hbm.at[s], srcv, gsem.at[0]).wait()
        pltpu.make_async_copy(dst_hbm.at[s], dstv, gsem.at[1]).wait()
        for z in range(ZROWS // CHUNK):
            pltpu.make_async_copy(
                zbuf, acc.at[pl.ds(s * ZROWS + z * CHUNK, CHUNK)],
                ssem.at[z]).wait()
        plsc.subcore_barrier()

        def hop(u_hbm):
            usrc = u_hbm.at[c]
            for kk in range(NDEPTH):
                pltpu.async_copy(usrc.at[srcv.at[kk]], rbuf.at[kk],
                                 gsem.at[kk])

            @pl.loop(0, CH, step=NBUF)
            def _(j):
                for l in range(NBUF):
                    k = j + l
                    pltpu.make_async_copy(usrc.at[srcv.at[k]], rbuf.at[l],
                                          gsem.at[l]).wait()
                    pltpu.async_copy(rbuf.at[l], acc.at[dstv.at[k]],
                                     ssem.at[l], add=True)
                    la = (l + NDEPTH) % NBUF
                    ka = k + NDEPTH

                    @pl.when(ka < CH)
                    def _():
                        @pl.when(ka >= NBUF)
                        def _():
                            pltpu.make_async_copy(
                                rbuf.at[la], acc.at[dstv.at[k]],
                                ssem.at[la]).wait()

                        pltpu.async_copy(usrc.at[srcv.at[ka]], rbuf.at[la],
                                         gsem.at[la])

            for l in range(NBUF):
                pltpu.make_async_copy(rbuf.at[l],
                                      acc.at[dstv.at[CH - NBUF + l]],
                                      ssem.at[l]).wait()
            plsc.subcore_barrier()

        def combine(q_hbm, unext_hbm):
            offs = []
            off = s * CROWS
            for sz in CSPLIT:
                offs.append((off, sz))
                off += sz
            for idx, (off, sz) in enumerate(offs):
                ab = abuf.at[idx % 2]
                if idx >= 2:
                    poff, psz = offs[idx - 2]
                    pltpu.make_async_copy(ab.at[pl.ds(0, psz)],
                                          unext_hbm.at[c, pl.ds(poff, psz)],
                                          ssem.at[idx - 2]).wait()
                pltpu.async_copy(acc.at[pl.ds(off, sz)], ab.at[pl.ds(0, sz)],
                                 gsem.at[0])
                pltpu.async_copy(q_hbm.at[c, pl.ds(off, sz)],
                                 qbuf.at[pl.ds(0, sz)], gsem.at[1])
                pltpu.async_copy(d2_hbm.at[pl.ds(off, sz)],
                                 dbuf.at[pl.ds(0, sz)], gsem.at[2])
                pltpu.make_async_copy(acc.at[pl.ds(off, sz)],
                                      ab.at[pl.ds(0, sz)], gsem.at[0]).wait()
                pltpu.make_async_copy(q_hbm.at[c, pl.ds(off, sz)],
                                      qbuf.at[pl.ds(0, sz)], gsem.at[1]).wait()
                pltpu.make_async_copy(d2_hbm.at[pl.ds(off, sz)],
                                      dbuf.at[pl.ds(0, sz)], gsem.at[2]).wait()

                @pl.loop(0, sz)
                def _(i):
                    ab[i, :] = qbuf[i, :] + dbuf[i, :] * ab[i, :]

                pltpu.async_copy(ab.at[pl.ds(0, sz)],
                                 unext_hbm.at[c, pl.ds(off, sz)], ssem.at[idx])
                pltpu.async_copy(zbuf.at[pl.ds(0, sz)], acc.at[pl.ds(off, sz)],
                                 ssem.at[5 + idx])
            for idx, (off, sz) in enumerate(offs):
                if idx >= len(offs) - 2:
                    pltpu.make_async_copy(abuf.at[idx % 2].at[pl.ds(0, sz)],
                                          unext_hbm.at[c, pl.ds(off, sz)],
                                          ssem.at[idx]).wait()
                pltpu.make_async_copy(zbuf.at[pl.ds(0, sz)],
                                      acc.at[pl.ds(off, sz)],
                                      ssem.at[5 + idx]).wait()
            plsc.subcore_barrier()

        hop(u0_hbm)
        combine(qa_hbm, u1_hbm)
        hop(u1_hbm)
        combine(qb_hbm, u2_hbm)
        hop(u2_hbm)

        for z in range(ZROWS // CHUNK):
            sl = pl.ds(s * ZROWS + z * CHUNK, CHUNK)
            pltpu.async_copy(acc.at[sl], raw_hbm.at[c, sl], ssem.at[z])
        for z in range(ZROWS // CHUNK):
            sl = pl.ds(s * ZROWS + z * CHUNK, CHUNK)
            pltpu.make_async_copy(acc.at[sl], raw_hbm.at[c, sl],
                                  ssem.at[z]).wait()

    return layer_kernel(u0, qa, qb, d2, src3, dst3)[0]


BROW = 2000
BGRID = N // BROW


def _split16(dis, P, F, k, rows):
    fh = F // 2
    blocks = []
    for c in range(2):
        blk = dis * P[:, k * F + c * fh:k * F + (c + 1) * fh]
        if fh < FH:
            blk = jnp.concatenate(
                [blk, jnp.zeros((rows, FH - fh), jnp.float32)], axis=1)
        blocks.append(blk)
    return jnp.stack(blocks)


def _tc_pre(degp, x, W1p):
    def body(d_ref, x_ref, w_ref, p_ref, dis_ref, d2_ref,
             u0_ref, qa_ref, qb_ref):
        deg = d_ref[0, :, 0:1] + d_ref[1, :, 0:1]
        dis = jnp.where(deg > 0.0,
                        lax.rsqrt(jnp.maximum(deg, 1e-12)),
                        0.0)
        dis_ref[...] = dis
        d2_ref[...] = jnp.broadcast_to(dis * dis, (BROW, FH))
        P1 = jnp.dot(x_ref[...], w_ref[...],
                     preferred_element_type=jnp.float32, precision=_PREC)
        p_ref[...] = P1
        u0_ref[...] = _split16(dis, P1, 32, 3, BROW)
        qa_ref[...] = _split16(dis, P1, 32, 2, BROW)
        qb_ref[...] = _split16(dis, P1, 32, 1, BROW)

    return pl.pallas_call(
        body,
        grid=(BGRID,),
        in_specs=[
            pl.BlockSpec((2, BROW, 8), lambda i: (0, i, 0)),
            pl.BlockSpec((BROW, 128), lambda i: (i, 0)),
            pl.BlockSpec((128, 128), lambda i: (0, 0)),
        ],
        out_specs=(
            pl.BlockSpec((BROW, 128), lambda i: (i, 0)),
            pl.BlockSpec((BROW, 1), lambda i: (i, 0)),
            pl.BlockSpec((BROW, FH), lambda i: (i, 0)),
            pl.BlockSpec((2, BROW, FH), lambda i: (0, i, 0)),
            pl.BlockSpec((2, BROW, FH), lambda i: (0, i, 0)),
            pl.BlockSpec((2, BROW, FH), lambda i: (0, i, 0)),
        ),
        out_shape=(jax.ShapeDtypeStruct((N, 128), jnp.float32),
                   jax.ShapeDtypeStruct((N, 1), jnp.float32),
                   jax.ShapeDtypeStruct((N, FH), jnp.float32),
                   jax.ShapeDtypeStruct((NCORE, N, FH), jnp.float32),
                   jax.ShapeDtypeStruct((NCORE, N, FH), jnp.float32),
                   jax.ShapeDtypeStruct((NCORE, N, FH), jnp.float32)),
    )(degp, x, W1p)


def _tc_layer(raw, P, dis, b, Wnext, F, Fn):
    fh = F // 2
    Fp, Fo = Wnext.shape

    def body(a_ref, p_ref, d_ref, b_ref, w_ref,
             pn_ref, u0_ref, qa_ref, qb_ref):
        dis = d_ref[...]
        t = jnp.concatenate(
            [p_ref[:, 0:fh] + dis * a_ref[0, :, 0:fh],
             p_ref[:, fh:F] + dis * a_ref[1, :, 0:fh]], axis=1)
        h = jnp.maximum(t + b_ref[...], 0.0)
        pn = jnp.dot(h, w_ref[...], preferred_element_type=jnp.float32,
                     precision=_PREC)
        pn_ref[...] = pn
        u0_ref[...] = _split16(dis, pn, Fn, 3, BROW)
        qa_ref[...] = _split16(dis, pn, Fn, 2, BROW)
        qb_ref[...] = _split16(dis, pn, Fn, 1, BROW)

    Fin = P.shape[1]
    return pl.pallas_call(
        body,
        grid=(BGRID,),
        in_specs=[
            pl.BlockSpec((2, BROW, FH), lambda i: (0, i, 0)),
            pl.BlockSpec((BROW, Fin), lambda i: (i, 0)),
            pl.BlockSpec((BROW, 1), lambda i: (i, 0)),
            pl.BlockSpec((1, Fp), lambda i: (0, 0)),
            pl.BlockSpec((Fp, Fo), lambda i: (0, 0)),
        ],
        out_specs=(
            pl.BlockSpec((BROW, Fo), lambda i: (i, 0)),
            pl.BlockSpec((2, BROW, FH), lambda i: (0, i, 0)),
            pl.BlockSpec((2, BROW, FH), lambda i: (0, i, 0)),
            pl.BlockSpec((2, BROW, FH), lambda i: (0, i, 0)),
        ),
        out_shape=(jax.ShapeDtypeStruct((N, Fo), jnp.float32),
                   jax.ShapeDtypeStruct((NCORE, N, FH), jnp.float32),
                   jax.ShapeDtypeStruct((NCORE, N, FH), jnp.float32),
                   jax.ShapeDtypeStruct((NCORE, N, FH), jnp.float32)),
    )(raw, P, dis, b.reshape(1, -1), Wnext)


def _tc_layer23(raw, P2, dis, b2, W3p):
    def body(a_ref, p_ref, d_ref, b_ref, w_ref,
             pn_ref, u0_ref, qa_ref, qb_ref):
        dis = d_ref[...]
        t = jnp.concatenate(
            [p_ref[:, 0:8] + dis * a_ref[0, :, 0:8],
             p_ref[:, 8:16] + dis * a_ref[1, :, 0:8]], axis=1)
        h = jnp.maximum(t + b_ref[...], 0.0)
        pn = jnp.dot(h, w_ref[...], preferred_element_type=jnp.float32,
                     precision=_PREC)
        pn_ref[...] = pn

        def dup(k):
            blk = jnp.concatenate(
                [dis * pn[:, 2 * k:2 * k + 2],
                 jnp.zeros((BROW, FH - 2), jnp.float32)], axis=1)
            return jnp.stack([blk, blk])

        u0_ref[...] = dup(3)
        qa_ref[...] = dup(2)
        qb_ref[...] = dup(1)

    return pl.pallas_call(
        body,
        grid=(BGRID,),
        in_specs=[
            pl.BlockSpec((2, BROW, FH), lambda i: (0, i, 0)),
            pl.BlockSpec((BROW, 64), lambda i: (i, 0)),
            pl.BlockSpec((BROW, 1), lambda i: (i, 0)),
            pl.BlockSpec((1, 16), lambda i: (0, 0)),
            pl.BlockSpec((16, 8), lambda i: (0, 0)),
        ],
        out_specs=(
            pl.BlockSpec((BROW, 8), lambda i: (i, 0)),
            pl.BlockSpec((2, BROW, FH), lambda i: (0, i, 0)),
            pl.BlockSpec((2, BROW, FH), lambda i: (0, i, 0)),
            pl.BlockSpec((2, BROW, FH), lambda i: (0, i, 0)),
        ),
        out_shape=(jax.ShapeDtypeStruct((N, 8), jnp.float32),
                   jax.ShapeDtypeStruct((NCORE, N, FH), jnp.float32),
                   jax.ShapeDtypeStruct((NCORE, N, FH), jnp.float32),
                   jax.ShapeDtypeStruct((NCORE, N, FH), jnp.float32)),
    )(raw, P2, dis, b2.reshape(1, -1), W3p)


def _tc_final(raw, P3, dis, b3):
    def body(a_ref, p_ref, d_ref, b_ref, o_ref):
        z = (p_ref[:, 0:2] + d_ref[...] * a_ref[0, :N, 0:2] + b_ref[...])
        m = jnp.max(z, axis=1, keepdims=True)
        lse = m + jnp.log(jnp.sum(jnp.exp(z - m), axis=1, keepdims=True))
        o_ref[...] = z - lse

    return pl.pallas_call(
        body, out_shape=jax.ShapeDtypeStruct((N, 2), jnp.float32),
    )(raw, P3, dis, b3.reshape(1, -1))


def kernel(x, edge_index, W1, b1, W2, b2, W3, b3):
    src = edge_index[0].astype(jnp.int32)
    dst = edge_index[1].astype(jnp.int32)
    pad = E_PAD - E
    src3 = jnp.concatenate([src, jnp.zeros((pad,), jnp.int32)]).reshape(
        NSUB, CH, CHUNK)
    dst3 = jnp.concatenate([dst, jnp.full((pad,), N, jnp.int32)]).reshape(
        NSUB, CH, CHUNK)
    src3d = src3.reshape(NSUB * 2, CH // 2, CHUNK)
    dst3d = dst3.reshape(NSUB * 2, CH // 2, CHUNK)
    del src3d

    ones8 = jnp.ones((CHUNK, 8), jnp.float32)
    z8 = jnp.zeros((ZROWS, 8), jnp.float32)

    W1p = jnp.concatenate([W1[i * 128:(i + 1) * 128] for i in range(4)], axis=1)
    W2p = jnp.concatenate([W2[i * 32:(i + 1) * 32] for i in range(4)], axis=1)
    W3p = jnp.concatenate([W3[i * 16:(i + 1) * 16] for i in range(4)], axis=1)

    degp = _deg_partials(dst3d, ones8, z8)
    P1, dis, d2, u0, qa, qb = _tc_pre(degp, x, W1p)

    raw1 = _layer_sc(u0, qa, qb, d2, src3, dst3)
    P2, u0, qa, qb = _tc_layer(raw1, P1, dis, b1, W2p, 32, 16)

    raw2 = _layer_sc(u0, qa, qb, d2, src3, dst3)
    P3, u0, qa, qb = _tc_layer23(raw2, P2, dis, b2, W3p)

    raw3 = _layer_sc(u0, qa, qb, d2, src3, dst3)
    return _tc_final(raw3, P3, dis, b3)


# speedup vs baseline: 1.2979x; 1.0007x over previous
import functools

import jax
import jax.numpy as jnp
from jax import lax
from jax.experimental import pallas as pl
from jax.experimental.pallas import tpu as pltpu
from jax.experimental.pallas import tpu_sc as plsc

N = 10000
E = 320000
NSUB = 16
NCORE = 2
CHUNK = 128
EPS = 20480
E_PAD = NSUB * EPS
CH = EPS // CHUNK
FH = 16
N_ACC = 10240
ZROWS = N_ACC // NSUB
CROWS = N // NSUB
CSPLIT = (128, 128, 128, 128, 113)
NBUF = 10
NDEPTH = 8

_PREC = jax.lax.Precision.HIGHEST


def _mesh():
    return plsc.VectorSubcoreMesh(core_axis_name="c", subcore_axis_name="s")


_SC_PARAMS = pltpu.CompilerParams(use_tc_tiling_on_sc=False)


def _deg_partials(dst3, ones_rows, zrows):
    @functools.partial(
        pl.kernel,
        out_type=jax.ShapeDtypeStruct((NCORE, N_ACC, 8), jnp.float32),
        mesh=_mesh(),
        scratch_types=[
            pltpu.VMEM((CH // 2, CHUNK), jnp.int32),
            pltpu.VMEM((CHUNK, 8), jnp.float32),
            pltpu.VMEM_SHARED((N_ACC, 8), jnp.float32),
            pltpu.SemaphoreType.DMA((4,)),
        ],
        compiler_params=_SC_PARAMS,
    )
    def deg_kernel(dst_hbm, ones_hbm, z_hbm, out_hbm, dstv, onesv, acc, sem):
        c = lax.axis_index("c")
        s = lax.axis_index("s")
        w = c * NSUB + s
        pltpu.async_copy(z_hbm, acc.at[pl.ds(s * ZROWS, ZROWS)], sem.at[0])
        pltpu.async_copy(dst_hbm.at[w], dstv, sem.at[1])
        pltpu.async_copy(ones_hbm, onesv, sem.at[2])
        pltpu.make_async_copy(z_hbm, acc.at[pl.ds(s * ZROWS, ZROWS)],
                              sem.at[0]).wait()
        pltpu.make_async_copy(dst_hbm.at[w], dstv, sem.at[1]).wait()
        pltpu.make_async_copy(ones_hbm, onesv, sem.at[2]).wait()
        plsc.subcore_barrier()

        @pl.loop(0, CH // 2, step=4)
        def _(j):
            for l in range(4):
                pltpu.async_copy(onesv, acc.at[dstv.at[j + l]], sem.at[l],
                                 add=True)
            for l in range(4):
                pltpu.make_async_copy(onesv, acc.at[dstv.at[j + l]],
                                      sem.at[l]).wait()

        plsc.subcore_barrier()
        pltpu.sync_copy(acc.at[pl.ds(s * ZROWS, ZROWS)],
                        out_hbm.at[c, pl.ds(s * ZROWS, ZROWS)])

    return deg_kernel(dst3, ones_rows, zrows)


def _layer_sc(u0, qa, qb, d2, src3, dst3):
    @functools.partial(
        pl.kernel,
        out_type=(jax.ShapeDtypeStruct((NCORE, N_ACC, FH), jnp.float32),
                  jax.ShapeDtypeStruct((NCORE, N, FH), jnp.float32),
                  jax.ShapeDtypeStruct((NCORE, N, FH), jnp.float32)),
        mesh=_mesh(),
        scratch_types=[
            pltpu.VMEM((CH, CHUNK), jnp.int32),
            pltpu.VMEM((CH, CHUNK), jnp.int32),
            pltpu.VMEM((NBUF, CHUNK, FH), jnp.float32),
            pltpu.VMEM((2, CHUNK, FH), jnp.float32),
            pltpu.VMEM((CHUNK, FH), jnp.float32),
            pltpu.VMEM((CHUNK, FH), jnp.float32),
            pltpu.VMEM((CHUNK, FH), jnp.float32),
            pltpu.VMEM_SHARED((N_ACC, FH), jnp.float32),
            pltpu.SemaphoreType.DMA((NBUF,)),
            pltpu.SemaphoreType.DMA((NBUF,)),
        ],
        compiler_params=_SC_PARAMS,
    )
    def layer_kernel(u0_hbm, qa_hbm, qb_hbm, d2_hbm, src_hbm, dst_hbm,
                     raw_hbm, u1_hbm, u2_hbm,
                     srcv, dstv, rbuf, abuf, qbuf, dbuf, zbuf,
                     acc, gsem, ssem):
        c = lax.axis_index("c")
        s = lax.axis_index("s")

        pltpu.async_copy(src_hbm.at[s], srcv, gsem.at[0])
        pltpu.async_copy(dst_hbm.at[s], dstv, gsem.at[1])

        @pl.loop(0, CHUNK)
        def _(i):
            zbuf[i, :] = jnp.zeros((FH,), jnp.float32)

        for z in range(ZROWS // CHUNK):
            pltpu.async_copy(zbuf, acc.at[pl.ds(s * ZROWS + z * CHUNK, CHUNK)],
                             ssem.at[z])
        pltpu.make_async_copy(src_hbm.at[s], srcv, gsem.at[0]).wait()
        pltpu.make_async_copy(dst_hbm.at[s], dstv, gsem.at[1]).wait()
        for z in range(ZROWS // CHUNK):
            pltpu.make_async_copy(
                zbuf, acc.at[pl.ds(s * ZROWS + z * CHUNK, CHUNK)],
                ssem.at[z]).wait()
        plsc.subcore_barrier()

        def hop(u_hbm):
            usrc = u_hbm.at[c]
            for kk in range(NDEPTH):
                pltpu.async_copy(usrc.at[srcv.at[kk]], rbuf.at[kk],
                                 gsem.at[kk])

            @pl.loop(0, CH, step=NBUF)
            def _(j):
                for l in range(NBUF):
                    k = j + l
                    pltpu.make_async_copy(usrc.at[srcv.at[k]], rbuf.at[l],
                                          gsem.at[l]).wait()
                    pltpu.async_copy(rbuf.at[l], acc.at[dstv.at[k]],
                                     ssem.at[l], add=True)
                    la = (l + NDEPTH) % NBUF
                    ka = k + NDEPTH

                    @pl.when(ka < CH)
                    def _():
                        @pl.when(ka >= NBUF)
                        def _():
                            pltpu.make_async_copy(
                                rbuf.at[la], acc.at[dstv.at[k]],
                                ssem.at[la]).wait()

                        pltpu.async_copy(usrc.at[srcv.at[ka]], rbuf.at[la],
                                         gsem.at[la])

            for l in range(NBUF):
                pltpu.make_async_copy(rbuf.at[l],
                                      acc.at[dstv.at[CH - NBUF + l]],
                                      ssem.at[l]).wait()
            plsc.subcore_barrier()

        def combine(q_hbm, unext_hbm):
            offs = []
            off = s * CROWS
            for sz in CSPLIT:
                offs.append((off, sz))
                off += sz
            for idx, (off, sz) in enumerate(offs):
                ab = abuf.at[idx % 2]
                if idx >= 2:
                    poff, psz = offs[idx - 2]
                    pltpu.make_async_copy(ab.at[pl.ds(0, psz)],
                                          unext_hbm.at[c, pl.ds(poff, psz)],
                                          ssem.at[idx - 2]).wait()
                pltpu.async_copy(acc.at[pl.ds(off, sz)], ab.at[pl.ds(0, sz)],
                                 gsem.at[0])
                pltpu.async_copy(q_hbm.at[c, pl.ds(off, sz)],
                                 qbuf.at[pl.ds(0, sz)], gsem.at[1])
                pltpu.async_copy(d2_hbm.at[pl.ds(off, sz)],
                                 dbuf.at[pl.ds(0, sz)], gsem.at[2])
                pltpu.make_async_copy(acc.at[pl.ds(off, sz)],
                                      ab.at[pl.ds(0, sz)], gsem.at[0]).wait()
                pltpu.make_async_copy(q_hbm.at[c, pl.ds(off, sz)],
                                      qbuf.at[pl.ds(0, sz)], gsem.at[1]).wait()
                pltpu.make_async_copy(d2_hbm.at[pl.ds(off, sz)],
                                      dbuf.at[pl.ds(0, sz)], gsem.at[2]).wait()

                @pl.loop(0, sz)
                def _(i):
                    ab[i, :] = qbuf[i, :] + dbuf[i, :] * ab[i, :]

                pltpu.async_copy(ab.at[pl.ds(0, sz)],
                                 unext_hbm.at[c, pl.ds(off, sz)], ssem.at[idx])
                pltpu.async_copy(zbuf.at[pl.ds(0, sz)], acc.at[pl.ds(off, sz)],
                                 ssem.at[5 + idx])
            for idx, (off, sz) in enumerate(offs):
                if idx >= len(offs) - 2:
                    pltpu.make_async_copy(abuf.at[idx % 2].at[pl.ds(0, sz)],
                                          unext_hbm.at[c, pl.ds(off, sz)],
                                          ssem.at[idx]).wait()
                pltpu.make_async_copy(zbuf.at[pl.ds(0, sz)],
                                      acc.at[pl.ds(off, sz)],
                                      ssem.at[5 + idx]).wait()
            plsc.subcore_barrier()

        hop(u0_hbm)
        combine(qa_hbm, u1_hbm)
        hop(u1_hbm)
        combine(qb_hbm, u2_hbm)
        hop(u2_hbm)

        for z in range(ZROWS // CHUNK):
            sl = pl.ds(s * ZROWS + z * CHUNK, CHUNK)
            pltpu.async_copy(acc.at[sl], raw_hbm.at[c, sl], ssem.at[z])
        for z in range(ZROWS // CHUNK):
            sl = pl.ds(s * ZROWS + z * CHUNK, CHUNK)
            pltpu.make_async_copy(acc.at[sl], raw_hbm.at[c, sl],
                                  ssem.at[z]).wait()

    return layer_kernel(u0, qa, qb, d2, src3, dst3)[0]


BROW = 2000
BGRID = N // BROW


def _split16(dis, P, F, k, rows):
    fh = F // 2
    blocks = []
    for c in range(2):
        blk = dis * P[:, k * F + c * fh:k * F + (c + 1) * fh]
        if fh < FH:
            blk = jnp.concatenate(
                [blk, jnp.zeros((rows, FH - fh), jnp.float32)], axis=1)
        blocks.append(blk)
    return jnp.stack(blocks)


def _tc_pre(degp, x, W1p):
    def body(d_ref, x_ref, w_ref, p_ref, dis_ref, d2_ref,
             u0_ref, qa_ref, qb_ref):
        deg = d_ref[0, :, 0:1] + d_ref[1, :, 0:1]
        dis = jnp.where(deg > 0.0,
                        lax.rsqrt(jnp.maximum(deg, 1e-12)),
                        0.0)
        dis_ref[...] = dis
        d2_ref[...] = jnp.broadcast_to(dis * dis, (BROW, FH))
        P1 = jnp.dot(x_ref[...], w_ref[...],
                     preferred_element_type=jnp.float32, precision=_PREC)
        p_ref[...] = P1
        u0_ref[...] = _split16(dis, P1, 32, 3, BROW)
        qa_ref[...] = _split16(dis, P1, 32, 2, BROW)
        qb_ref[...] = _split16(dis, P1, 32, 1, BROW)

    return pl.pallas_call(
        body,
        grid=(BGRID,),
        in_specs=[
            pl.BlockSpec((2, BROW, 8), lambda i: (0, i, 0)),
            pl.BlockSpec((BROW, 128), lambda i: (i, 0)),
            pl.BlockSpec((128, 128), lambda i: (0, 0)),
        ],
        out_specs=(
            pl.BlockSpec((BROW, 128), lambda i: (i, 0)),
            pl.BlockSpec((BROW, 1), lambda i: (i, 0)),
            pl.BlockSpec((BROW, FH), lambda i: (i, 0)),
            pl.BlockSpec((2, BROW, FH), lambda i: (0, i, 0)),
            pl.BlockSpec((2, BROW, FH), lambda i: (0, i, 0)),
            pl.BlockSpec((2, BROW, FH), lambda i: (0, i, 0)),
        ),
        out_shape=(jax.ShapeDtypeStruct((N, 128), jnp.float32),
                   jax.ShapeDtypeStruct((N, 1), jnp.float32),
                   jax.ShapeDtypeStruct((N, FH), jnp.float32),
                   jax.ShapeDtypeStruct((NCORE, N, FH), jnp.float32),
                   jax.ShapeDtypeStruct((NCORE, N, FH), jnp.float32),
                   jax.ShapeDtypeStruct((NCORE, N, FH), jnp.float32)),
    )(degp, x, W1p)


def _tc_layer(raw, P, dis, b, Wnext, F, Fn):
    fh = F // 2
    Fp, Fo = Wnext.shape

    def body(a_ref, p_ref, d_ref, b_ref, w_ref,
             pn_ref, u0_ref, qa_ref, qb_ref):
        dis = d_ref[...]
        t = jnp.concatenate(
            [p_ref[:, 0:fh] + dis * a_ref[0, :, 0:fh],
             p_ref[:, fh:F] + dis * a_ref[1, :, 0:fh]], axis=1)
        h = jnp.maximum(t + b_ref[...], 0.0)
        pn = jnp.dot(h, w_ref[...], preferred_element_type=jnp.float32,
                     precision=_PREC)
        pn_ref[...] = pn
        u0_ref[...] = _split16(dis, pn, Fn, 3, BROW)
        qa_ref[...] = _split16(dis, pn, Fn, 2, BROW)
        qb_ref[...] = _split16(dis, pn, Fn, 1, BROW)

    Fin = P.shape[1]
    return pl.pallas_call(
        body,
        grid=(BGRID,),
        in_specs=[
            pl.BlockSpec((2, BROW, FH), lambda i: (0, i, 0)),
            pl.BlockSpec((BROW, Fin), lambda i: (i, 0)),
            pl.BlockSpec((BROW, 1), lambda i: (i, 0)),
            pl.BlockSpec((1, Fp), lambda i: (0, 0)),
            pl.BlockSpec((Fp, Fo), lambda i: (0, 0)),
        ],
        out_specs=(
            pl.BlockSpec((BROW, Fo), lambda i: (i, 0)),
            pl.BlockSpec((2, BROW, FH), lambda i: (0, i, 0)),
            pl.BlockSpec((2, BROW, FH), lambda i: (0, i, 0)),
            pl.BlockSpec((2, BROW, FH), lambda i: (0, i, 0)),
        ),
        out_shape=(jax.ShapeDtypeStruct((N, Fo), jnp.float32),
                   jax.ShapeDtypeStruct((NCORE, N, FH), jnp.float32),
                   jax.ShapeDtypeStruct((NCORE, N, FH), jnp.float32),
                   jax.ShapeDtypeStruct((NCORE, N, FH), jnp.float32)),
    )(raw, P, dis, b.reshape(1, -1), Wnext)


def _tc_layer23(raw, P2, dis, b2, W3p):
    def body(a_ref, p_ref, d_ref, b_ref, w_ref,
             pn_ref, u0_ref, qa_ref, qb_ref):
        dis = d_ref[...]
        t = jnp.concatenate(
            [p_ref[:, 0:8] + dis * a_ref[0, :, 0:8],
             p_ref[:, 8:16] + dis * a_ref[1, :, 0:8]], axis=1)
        h = jnp.maximum(t + b_ref[...], 0.0)
        pn = jnp.dot(h, w_ref[...], preferred_element_type=jnp.float32,
                     precision=_PREC)
        pn_ref[...] = pn

        def dup(k):
            blk = jnp.concatenate(
                [dis * pn[:, 2 * k:2 * k + 2],
                 jnp.zeros((BROW, FH - 2), jnp.float32)], axis=1)
            return jnp.stack([blk, blk])

        u0_ref[...] = dup(3)
        qa_ref[...] = dup(2)
        qb_ref[...] = dup(1)

    return pl.pallas_call(
        body,
        grid=(BGRID,),
        in_specs=[
            pl.BlockSpec((2, BROW, FH), lambda i: (0, i, 0)),
            pl.BlockSpec((BROW, 64), lambda i: (i, 0)),
            pl.BlockSpec((BROW, 1), lambda i: (i, 0)),
            pl.BlockSpec((1, 16), lambda i: (0, 0)),
            pl.BlockSpec((16, 8), lambda i: (0, 0)),
        ],
        out_specs=(
            pl.BlockSpec((BROW, 8), lambda i: (i, 0)),
            pl.BlockSpec((2, BROW, FH), lambda i: (0, i, 0)),
            pl.BlockSpec((2, BROW, FH), lambda i: (0, i, 0)),
            pl.BlockSpec((2, BROW, FH), lambda i: (0, i, 0)),
        ),
        out_shape=(jax.ShapeDtypeStruct((N, 8), jnp.float32),
                   jax.ShapeDtypeStruct((NCORE, N, FH), jnp.float32),
                   jax.ShapeDtypeStruct((NCORE, N, FH), jnp.float32),
                   jax.ShapeDtypeStruct((NCORE, N, FH), jnp.float32)),
    )(raw, P2, dis, b2.reshape(1, -1), W3p)


def _tc_final(raw, P3, dis, b3):
    def body(a_ref, p_ref, d_ref, b_ref, o_ref):
        z = (p_ref[:, 0:2] + d_ref[...] * a_ref[0, :N, 0:2] + b_ref[...])
        m = jnp.max(z, axis=1, keepdims=True)
        lse = m + jnp.log(jnp.sum(jnp.exp(z - m), axis=1, keepdims=True))
        o_ref[...] = z - lse

    return pl.pallas_call(
        body, out_shape=jax.ShapeDtypeStruct((N, 2), jnp.float32),
    )(raw, P3, dis, b3.reshape(1, -1))


def kernel(x, edge_index, W1, b1, W2, b2, W3, b3):
    src = edge_index[0].astype(jnp.int32)
    dst = edge_index[1].astype(jnp.int32)
    pad = E_PAD - E
    src3 = jnp.concatenate([src, jnp.zeros((pad,), jnp.int32)]).reshape(
        NSUB, CH, CHUNK)
    dst3 = jnp.concatenate([dst, jnp.full((pad,), N, jnp.int32)]).reshape(
        NSUB, CH, CHUNK)
    dst3d = dst3.reshape(NSUB * 2, CH // 2, CHUNK)

    ones8 = jnp.ones((CHUNK, 8), jnp.float32)
    z8 = jnp.zeros((ZROWS, 8), jnp.float32)

    W1p = jnp.concatenate([W1[i * 128:(i + 1) * 128] for i in range(4)], axis=1)
    W2p = jnp.concatenate([W2[i * 32:(i + 1) * 32] for i in range(4)], axis=1)
    W3p = jnp.concatenate([W3[i * 16:(i + 1) * 16] for i in range(4)], axis=1)

    degp = _deg_partials(dst3d, ones8, z8)
    P1, dis, d2, u0, qa, qb = _tc_pre(degp, x, W1p)

    raw1 = _layer_sc(u0, qa, qb, d2, src3, dst3)
    P2, u0, qa, qb = _tc_layer(raw1, P1, dis, b1, W2p, 32, 16)

    raw2 = _layer_sc(u0, qa, qb, d2, src3, dst3)
    P3, u0, qa, qb = _tc_layer23(raw2, P2, dis, b2, W3p)

    raw3 = _layer_sc(u0, qa, qb, d2, src3, dst3)
    return _tc_final(raw3, P3, dis, b3)
